# baseline (speedup 1.0000x reference)
import jax
import jax.numpy as jnp
from jax.experimental import pallas as pl
from jax.experimental.pallas import tpu as pltpu

N = 10000
E = 160000
D = 256
H = 8
C = D // H
L = 6
ED = 4
OUT = 128
G = 64
NFREQ = D // 6


def _readout_body(g_ref, w1_ref, b1_ref, w2_ref, b2_ref, w3_ref, b3_ref, out_ref):
    g = g_ref[...]
    g = jax.nn.relu(jnp.dot(g, w1_ref[...], preferred_element_type=jnp.float32) + b1_ref[...])
    g = jax.nn.relu(jnp.dot(g, w2_ref[...], preferred_element_type=jnp.float32) + b2_ref[...])
    out_ref[...] = jnp.dot(g, w3_ref[...], preferred_element_type=jnp.float32) + b3_ref[...]


def kernel(x, edge_index, edge_attr, pos, batch, params):
    src = edge_index[0]
    dst = edge_index[1]
    h = x @ params['node_emb_W'] + params['node_emb_b']
    e = edge_attr @ params['edge_emb_W'] + params['edge_emb_b']
    freq = jnp.linspace(0.0, 10.0, NFREQ, dtype=jnp.float32)
    enc = []
    for i in range(3):
        ang = pos[:, i:i + 1] * freq[None, :]
        enc.append(jnp.sin(ang))
        enc.append(jnp.cos(ang))
    pe = jnp.concatenate(enc, axis=1) @ params['pos_W'] + params['pos_b']
    h = h + pe
    scale = 1.0 / (float(C) ** 0.5)
    for l in range(L):
        p = params['layers'][l]
        res = h
        q = (h @ p['Wq'] + p['bq'])[dst].reshape(E, H, C)
        ee = (e @ p['We'] + p['be']).reshape(E, H, C)
        k = (h @ p['Wk'] + p['bk'])[src].reshape(E, H, C) + ee
        v = (h @ p['Wv'] + p['bv'])[src].reshape(E, H, C) + ee
        logits = (q * k).sum(-1) * scale
        m = jax.ops.segment_max(logits, dst, num_segments=N)
        m = jnp.where(jnp.isfinite(m), m, 0.0)
        ex = jnp.exp(logits - m[dst])
        s = jax.ops.segment_sum(ex, dst, num_segments=N)
        alpha = ex / (s[dst] + 1e-16)
        msg = jax.ops.segment_sum((alpha[..., None] * v).reshape(E, H * C), dst, num_segments=N)
        r = h @ p['Wskip'] + p['bskip']
        beta = jax.nn.sigmoid(jnp.concatenate([msg, r, msg - r], axis=1) @ p['Wbeta'] + p['bbeta'])
        out = beta * r + (1.0 - beta) * msg
        mu = out.mean(-1, keepdims=True)
        var = out.var(-1, keepdims=True)
        out = (out - mu) / jnp.sqrt(var + 1e-5) * p['ln_g'] + p['ln_b']
        out = jax.nn.relu(out)
        if l > 0:
            out = out + res
        h = out
    add = jax.ops.segment_sum(h, batch, num_segments=G)
    cnt = jax.ops.segment_sum(jnp.ones((N, 1), jnp.float32), batch, num_segments=G)
    mean = add / jnp.maximum(cnt, 1.0)
    mx = jax.ops.segment_max(h, batch, num_segments=G)
    mx = jnp.where(jnp.isfinite(mx), mx, 0.0)
    g = jnp.concatenate([mean, mx, add], axis=1)
    out = pl.pallas_call(
        _readout_body,
        out_shape=jax.ShapeDtypeStruct((G, OUT), jnp.float32),
    )(g, params['r_W1'], params['r_b1'], params['r_W2'], params['r_b2'],
      params['r_W3'], params['r_b3'])
    return out


# speedup vs baseline: 1.0941x; 1.0941x over previous
import functools

import jax
import jax.numpy as jnp
from jax import lax
from jax.experimental import pallas as pl
from jax.experimental.pallas import tpu as pltpu
from jax.experimental.pallas import tpu_sc as plsc

N = 10000
E = 160000
D = 256
H = 8
C = D // H
L = 6
ED = 4
OUT = 128
G = 64
NFREQ = D // 6

NW = 32
GCH = 200
GITER = E // (NW * GCH)

_mesh = plsc.VectorSubcoreMesh(core_axis_name="c", subcore_axis_name="s")


def _gather_body(q_hbm, k_hbm, v_hbm, di_hbm, si_hbm, qo, ko, vo,
                 idxd, idxs, rows, sem):
    wid = lax.axis_index("c") * 16 + lax.axis_index("s")

    def chunk(i, carry):
        base = (i * NW + wid) * GCH
        pltpu.sync_copy(di_hbm.at[pl.ds(base, GCH)], idxd)
        pltpu.sync_copy(si_hbm.at[pl.ds(base, GCH)], idxs)
        pltpu.async_copy(q_hbm.at[idxd], rows, sem).wait()
        pltpu.sync_copy(rows, qo.at[pl.ds(base, GCH)])
        pltpu.async_copy(k_hbm.at[idxs], rows, sem).wait()
        pltpu.sync_copy(rows, ko.at[pl.ds(base, GCH)])
        pltpu.async_copy(v_hbm.at[idxs], rows, sem).wait()
        pltpu.sync_copy(rows, vo.at[pl.ds(base, GCH)])
        return carry

    lax.fori_loop(0, GITER, chunk, 0)


_gather = pl.kernel(
    _gather_body,
    out_type=[jax.ShapeDtypeStruct((E, D), jnp.float32)] * 3,
    mesh=_mesh,
    scratch_types=[
        pltpu.VMEM((GCH,), jnp.int32),
        pltpu.VMEM((GCH,), jnp.int32),
        pltpu.VMEM((GCH, D), jnp.float32),
        pltpu.SemaphoreType.DMA,
    ],
)


def _readout_body(g_ref, w1_ref, b1_ref, w2_ref, b2_ref, w3_ref, b3_ref, out_ref):
    g = g_ref[...]
    g = jax.nn.relu(jnp.dot(g, w1_ref[...], preferred_element_type=jnp.float32) + b1_ref[...])
    g = jax.nn.relu(jnp.dot(g, w2_ref[...], preferred_element_type=jnp.float32) + b2_ref[...])
    out_ref[...] = jnp.dot(g, w3_ref[...], preferred_element_type=jnp.float32) + b3_ref[...]


def kernel(x, edge_index, edge_attr, pos, batch, params):
    src = edge_index[0].astype(jnp.int32)
    dst = edge_index[1].astype(jnp.int32)
    perm = jnp.argsort(dst)
    dst_s = dst[perm]
    src_s = src[perm]
    ea_s = edge_attr[perm]

    h = x @ params['node_emb_W'] + params['node_emb_b']
    freq = jnp.linspace(0.0, 10.0, NFREQ, dtype=jnp.float32)
    enc = []
    for i in range(3):
        ang = pos[:, i:i + 1] * freq[None, :]
        enc.append(jnp.sin(ang))
        enc.append(jnp.cos(ang))
    pe = jnp.concatenate(enc, axis=1) @ params['pos_W'] + params['pos_b']
    h = h + pe

    EW = params['edge_emb_W']
    ebias = params['edge_emb_b']
    scale = 1.0 / (float(C) ** 0.5)
    for l in range(L):
        p = params['layers'][l]
        F = EW @ p['We']
        g = ebias @ p['We'] + p['be']
        res = h
        Q = h @ p['Wq'] + p['bq']
        K = h @ p['Wk'] + (p['bk'] + g)
        V = h @ p['Wv'] + (p['bv'] + g)
        qr, kr, vr = _gather(Q, K, V, dst_s, src_s)
        u = ea_s @ F
        kf = kr + u
        vf = vr + u
        logits = (qr * kf).reshape(E, H, C).sum(-1) * scale
        m = jax.ops.segment_max(logits, dst_s, num_segments=N)
        m = jnp.where(jnp.isfinite(m), m, 0.0)
        ex = jnp.exp(logits - m[dst_s])
        s = jax.ops.segment_sum(ex, dst_s, num_segments=N)
        alpha = ex / (s[dst_s] + 1e-16)
        msg = jax.ops.segment_sum((alpha[..., None] * vf.reshape(E, H, C)).reshape(E, D),
                                  dst_s, num_segments=N)
        r = h @ p['Wskip'] + p['bskip']
        beta = jax.nn.sigmoid(jnp.concatenate([msg, r, msg - r], axis=1) @ p['Wbeta'] + p['bbeta'])
        out = beta * r + (1.0 - beta) * msg
        mu = out.mean(-1, keepdims=True)
        var = out.var(-1, keepdims=True)
        out = (out - mu) / jnp.sqrt(var + 1e-5) * p['ln_g'] + p['ln_b']
        out = jax.nn.relu(out)
        if l > 0:
            out = out + res
        h = out

    add = jax.ops.segment_sum(h, batch, num_segments=G)
    cnt = jax.ops.segment_sum(jnp.ones((N, 1), jnp.float32), batch, num_segments=G)
    mean = add / jnp.maximum(cnt, 1.0)
    mx = jax.ops.segment_max(h, batch, num_segments=G)
    mx = jnp.where(jnp.isfinite(mx), mx, 0.0)
    gfeat = jnp.concatenate([mean, mx, add], axis=1)
    out = pl.pallas_call(
        _readout_body,
        out_shape=jax.ShapeDtypeStruct((G, OUT), jnp.float32),
    )(gfeat, params['r_W1'], params['r_b1'], params['r_W2'], params['r_b2'],
      params['r_W3'], params['r_b3'])
    return out


# speedup vs baseline: 1.3961x; 1.2760x over previous
import functools

import jax
import jax.numpy as jnp
from jax import lax
from jax.experimental import pallas as pl
from jax.experimental.pallas import tpu as pltpu
from jax.experimental.pallas import tpu_sc as plsc

N = 10000
E = 160000
D = 256
H = 8
C = D // H
L = 6
ED = 4
OUT = 128
G = 64
NFREQ = D // 6

NW = 32
GCH = 200
EPAD = 166400
GITER = EPAD // (NW * GCH)

NPW = 313
NPAD = NW * NPW
CE = 512
CE3 = 128
ACCW = NPW * D
MSW = 2512

_mesh = plsc.VectorSubcoreMesh(core_axis_name="c", subcore_axis_name="s")


def _gather_body(q_hbm, k_hbm, v_hbm, di_hbm, si_hbm, qo, ko, vo,
                 idxd, idxs, rows, sem):
    wid = lax.axis_index("c") * 16 + lax.axis_index("s")

    def chunk(i, carry):
        base = (i * NW + wid) * GCH
        pltpu.sync_copy(di_hbm.at[pl.ds(base, GCH)], idxd)
        pltpu.sync_copy(si_hbm.at[pl.ds(base, GCH)], idxs)
        pltpu.async_copy(q_hbm.at[idxd], rows, sem).wait()
        pltpu.sync_copy(rows, qo.at[pl.ds(base, GCH)])
        pltpu.async_copy(k_hbm.at[idxs], rows, sem).wait()
        pltpu.sync_copy(rows, ko.at[pl.ds(base, GCH)])
        pltpu.async_copy(v_hbm.at[idxs], rows, sem).wait()
        pltpu.sync_copy(rows, vo.at[pl.ds(base, GCH)])
        return carry

    lax.fori_loop(0, GITER, chunk, 0)


_gather = pl.kernel(
    _gather_body,
    out_type=[jax.ShapeDtypeStruct((EPAD, D), jnp.float32)] * 3,
    mesh=_mesh,
    scratch_types=[
        pltpu.VMEM((GCH,), jnp.int32),
        pltpu.VMEM((GCH,), jnp.int32),
        pltpu.VMEM((GCH, D), jnp.float32),
        pltpu.SemaphoreType.DMA,
    ],
)


def _extract48(ref, pos):
    total = jnp.float32(0)
    for j in range(3):
        v = ref[pl.ds(j * 16, 16)]
        lane = lax.broadcasted_iota(jnp.int32, (16,), 0) + (j * 16)
        total = total + jnp.sum(jnp.where(lane == pos, v, 0.0))
    return total.astype(jnp.int32)


def _soft_body(lg_hbm, vf_hbm, di_hbm, eb_hbm, msg_hbm,
               acc, msum, ssum, lbuf, dbuf, vbuf, abuf, ebv, sem):
    wid = lax.axis_index("c") * 16 + lax.axis_index("s")
    n0 = wid * NPW
    iota = lax.broadcasted_iota(jnp.int32, (16,), 0)
    half = lax.shift_right_logical(iota, 3)
    lane7 = jnp.bitwise_and(iota, 7)
    n0f = jnp.float32(n0)
    pltpu.sync_copy(eb_hbm, ebv)
    e0 = _extract48(ebv, wid)
    e1 = _extract48(ebv, wid + 1)
    estart = (e0 // 8) * 8
    nch = (e1 - estart + CE - 1) // CE
    nch3 = (e1 - estart + CE3 - 1) // CE3

    zz = jnp.zeros((16,), jnp.float32)

    def _zacc(j, c):
        acc[pl.ds(j * 16, 16)] = zz
        return c
    lax.fori_loop(0, ACCW // 16, _zacc, 0)

    def _zs(j, c):
        ssum[pl.ds(j * 16, 16)] = zz
        return c
    lax.fori_loop(0, MSW // 16, _zs, 0)

    def _edge_vec(base, r):
        lg = lbuf[pl.ds(r * 16, 16)]
        eg = base + 2 * r + half
        mk = (eg >= e0) & (eg < e1)
        dv = plsc.load_gather(dbuf, [2 * r + half], mask=mk)
        dl = jnp.clip(dv - n0f, 0.0, float(NPW - 1))
        fidx = (dl * 8.0).astype(jnp.int32) + lane7
        return lg, fidx, mk

    def p1_chunk(c, carry):
        base = estart + c * CE
        pltpu.sync_copy(lg_hbm.at[pl.ds(base * 8, CE * 8)], lbuf)
        pltpu.sync_copy(di_hbm.at[pl.ds(base, CE)], dbuf)

        def it(r, cc):
            lg, fidx, mk = _edge_vec(base, r)
            plsc.store_scatter(msum, [fidx], lg, mask=mk)
            return cc
        lax.fori_loop(0, CE // 2, it, 0)
        return carry
    lax.fori_loop(0, nch, p1_chunk, 0)

    def p2_chunk(c, carry):
        base = estart + c * CE
        pltpu.sync_copy(lg_hbm.at[pl.ds(base * 8, CE * 8)], lbuf)
        pltpu.sync_copy(di_hbm.at[pl.ds(base, CE)], dbuf)

        def it(r, cc):
            lg, fidx, mk = _edge_vec(base, r)
            m = plsc.load_gather(msum, [fidx], mask=mk)
            ex = jnp.exp(lg - m)
            plsc.addupdate_scatter(ssum, [fidx], ex, mask=mk & (iota < 8))
            plsc.addupdate_scatter(ssum, [fidx], ex, mask=mk & (iota >= 8))
            return cc
        lax.fori_loop(0, CE // 2, it, 0)
        return carry
    lax.fori_loop(0, nch, p2_chunk, 0)

    def p3_chunk(c, carry):
        base = estart + c * CE3
        pltpu.sync_copy(lg_hbm.at[pl.ds(base * 8, CE3 * 8)], lbuf.at[pl.ds(0, CE3 * 8)])
        pltpu.sync_copy(di_hbm.at[pl.ds(base, CE3)], dbuf.at[pl.ds(0, CE3)])
        pltpu.sync_copy(vf_hbm.at[pl.ds(base * D, CE3 * D)], vbuf)

        def ita(r, cc):
            lg, fidx, mk = _edge_vec(base, r)
            m = plsc.load_gather(msum, [fidx], mask=mk)
            s = plsc.load_gather(ssum, [fidx], mask=mk)
            abuf[pl.ds(r * 16, 16)] = jnp.exp(lg - m) / (s + 1e-16)
            return cc
        lax.fori_loop(0, CE3 * 8 // 16, ita, 0)

        def ite(e, cc):
            eg = jnp.full((16,), base + e, jnp.int32)
            mk = (eg >= e0) & (eg < e1)
            dv = plsc.load_gather(dbuf, [jnp.full((16,), e, jnp.int32)], mask=mk)
            bl = (jnp.clip(dv - n0f, 0.0, float(NPW - 1)) * float(D)).astype(jnp.int32) + iota
            for hh in range(H):
                av = plsc.load_gather(abuf, [jnp.full((16,), e * 8 + hh, jnp.int32)], mask=mk)
                for f in (2 * hh, 2 * hh + 1):
                    vv = vbuf[pl.ds(e * D + f * 16, 16)]
                    plsc.addupdate_scatter(acc, [bl + f * 16], vv * av, mask=mk)
            return cc
        lax.fori_loop(0, CE3, ite, 0)
        return carry
    lax.fori_loop(0, nch3, p3_chunk, 0)

    pltpu.sync_copy(acc, msg_hbm.at[pl.ds(n0 * D, NPW * D)])


_soft = pl.kernel(
    _soft_body,
    out_type=jax.ShapeDtypeStruct((NPAD * D,), jnp.float32),
    mesh=_mesh,
    compiler_params=pltpu.CompilerParams(needs_layout_passes=False),
    scratch_types=[
        pltpu.VMEM((ACCW,), jnp.float32),
        pltpu.VMEM((MSW,), jnp.float32),
        pltpu.VMEM((MSW,), jnp.float32),
        pltpu.VMEM((CE * 8,), jnp.float32),
        pltpu.VMEM((CE,), jnp.float32),
        pltpu.VMEM((CE3 * D,), jnp.float32),
        pltpu.VMEM((CE3 * 8,), jnp.float32),
        pltpu.VMEM((48,), jnp.float32),
        pltpu.SemaphoreType.DMA,
    ],
)


def _readout_body(g_ref, w1_ref, b1_ref, w2_ref, b2_ref, w3_ref, b3_ref, out_ref):
    g = g_ref[...]
    g = jax.nn.relu(jnp.dot(g, w1_ref[...], preferred_element_type=jnp.float32) + b1_ref[...])
    g = jax.nn.relu(jnp.dot(g, w2_ref[...], preferred_element_type=jnp.float32) + b2_ref[...])
    out_ref[...] = jnp.dot(g, w3_ref[...], preferred_element_type=jnp.float32) + b3_ref[...]


def kernel(x, edge_index, edge_attr, pos, batch, params):
    src = edge_index[0].astype(jnp.int32)
    dst = edge_index[1].astype(jnp.int32)
    perm = jnp.argsort(dst)
    dst_s = dst[perm]
    src_s = src[perm]
    ea_s = edge_attr[perm]
    ebounds = jnp.searchsorted(dst_s, jnp.arange(0, NPAD + NPW, NPW, dtype=jnp.int32)
                               ).astype(jnp.int32)
    ebounds = jnp.pad(ebounds, (0, 48 - NW - 1), constant_values=E)
    dst_s = jnp.pad(dst_s, (0, EPAD - E))
    src_s = jnp.pad(src_s, (0, EPAD - E))
    ea_s = jnp.pad(ea_s, ((0, EPAD - E), (0, 0)))
    dst_sf = dst_s.astype(jnp.float32)
    ebounds_f = ebounds.astype(jnp.float32)

    h = x @ params['node_emb_W'] + params['node_emb_b']
    freq = jnp.linspace(0.0, 10.0, NFREQ, dtype=jnp.float32)
    enc = []
    for i in range(3):
        ang = pos[:, i:i + 1] * freq[None, :]
        enc.append(jnp.sin(ang))
        enc.append(jnp.cos(ang))
    pe = jnp.concatenate(enc, axis=1) @ params['pos_W'] + params['pos_b']
    h = h + pe

    EW = params['edge_emb_W']
    ebias = params['edge_emb_b']
    scale = 1.0 / (float(C) ** 0.5)
    for l in range(L):
        p = params['layers'][l]
        F = EW @ p['We']
        g = ebias @ p['We'] + p['be']
        res = h
        Q = h @ p['Wq'] + p['bq']
        K = h @ p['Wk'] + (p['bk'] + g)
        V = h @ p['Wv'] + (p['bv'] + g)
        qr, kr, vr = _gather(Q, K, V, dst_s, src_s)
        u = ea_s @ F
        kf = kr + u
        vf = vr + u
        logits = (qr * kf).reshape(EPAD, H, C).sum(-1) * scale
        msg = _soft(logits.reshape(-1), vf.reshape(-1), dst_sf, ebounds_f)
        msg = msg.reshape(NPAD, D)[:N]
        r = h @ p['Wskip'] + p['bskip']
        beta = jax.nn.sigmoid(jnp.concatenate([msg, r, msg - r], axis=1) @ p['Wbeta'] + p['bbeta'])
        out = beta * r + (1.0 - beta) * msg
        mu = out.mean(-1, keepdims=True)
        var = out.var(-1, keepdims=True)
        out = (out - mu) / jnp.sqrt(var + 1e-5) * p['ln_g'] + p['ln_b']
        out = jax.nn.relu(out)
        if l > 0:
            out = out + res
        h = out

    add = jax.ops.segment_sum(h, batch, num_segments=G)
    cnt = jax.ops.segment_sum(jnp.ones((N, 1), jnp.float32), batch, num_segments=G)
    mean = add / jnp.maximum(cnt, 1.0)
    mx = jax.ops.segment_max(h, batch, num_segments=G)
    mx = jnp.where(jnp.isfinite(mx), mx, 0.0)
    gfeat = jnp.concatenate([mean, mx, add], axis=1)
    out = pl.pallas_call(
        _readout_body,
        out_shape=jax.ShapeDtypeStruct((G, OUT), jnp.float32),
    )(gfeat, params['r_W1'], params['r_b1'], params['r_W2'], params['r_b2'],
      params['r_W3'], params['r_b3'])
    return out


# speedup vs baseline: 1.6254x; 1.1642x over previous
import functools

import jax
import jax.numpy as jnp
from jax import lax
from jax.experimental import pallas as pl
from jax.experimental.pallas import tpu as pltpu
from jax.experimental.pallas import tpu_sc as plsc

N = 10000
E = 160000
D = 256
H = 8
C = D // H
L = 6
ED = 4
OUT = 128
G = 64
NFREQ = D // 6

NW = 32
GCH = 64
EPAD = 163840
EPW = EPAD // NW
GITER = EPW // (2 * GCH)

NPW = 313
NPAD = NW * NPW
CE = 512
CE3 = 128
ACCW = NPW * D
MSW = 2512

_mesh = plsc.VectorSubcoreMesh(core_axis_name="c", subcore_axis_name="s")


def _gather_body(q_hbm, k_hbm, v_hbm, di_hbm, si_hbm, qo, ko, vo,
                 idxd, idxs, qba, kba, vba, qbb, kbb, vbb,
                 sqa, ska, sva, sqb, skb, svb,
                 wqa, wka, wva, wqb, wkb, wvb):
    wid = lax.axis_index("c") * 16 + lax.axis_index("s")
    wbase = wid * EPW
    pltpu.sync_copy(di_hbm.at[pl.ds(wbase, EPW)], idxd)
    pltpu.sync_copy(si_hbm.at[pl.ds(wbase, EPW)], idxs)

    def it(i, carry):
        oa = 2 * i * GCH
        ob = oa + GCH
        ga = [
            pltpu.async_copy(q_hbm.at[idxd.at[pl.ds(oa, GCH)]], qba, sqa),
            pltpu.async_copy(k_hbm.at[idxs.at[pl.ds(oa, GCH)]], kba, ska),
            pltpu.async_copy(v_hbm.at[idxs.at[pl.ds(oa, GCH)]], vba, sva),
        ]
        gb = [
            pltpu.async_copy(q_hbm.at[idxd.at[pl.ds(ob, GCH)]], qbb, sqb),
            pltpu.async_copy(k_hbm.at[idxs.at[pl.ds(ob, GCH)]], kbb, skb),
            pltpu.async_copy(v_hbm.at[idxs.at[pl.ds(ob, GCH)]], vbb, svb),
        ]
        for g in ga:
            g.wait()
        wa = [
            pltpu.async_copy(qba, qo.at[pl.ds(wbase + oa, GCH)], wqa),
            pltpu.async_copy(kba, ko.at[pl.ds(wbase + oa, GCH)], wka),
            pltpu.async_copy(vba, vo.at[pl.ds(wbase + oa, GCH)], wva),
        ]
        for g in gb:
            g.wait()
        wb = [
            pltpu.async_copy(qbb, qo.at[pl.ds(wbase + ob, GCH)], wqb),
            pltpu.async_copy(kbb, ko.at[pl.ds(wbase + ob, GCH)], wkb),
            pltpu.async_copy(vbb, vo.at[pl.ds(wbase + ob, GCH)], wvb),
        ]
        for w in wa + wb:
            w.wait()
        return carry

    lax.fori_loop(0, GITER, it, 0)


_gather = pl.kernel(
    _gather_body,
    out_type=[jax.ShapeDtypeStruct((EPAD, D), jnp.float32)] * 3,
    mesh=_mesh,
    scratch_types=[
        pltpu.VMEM((EPW,), jnp.int32),
        pltpu.VMEM((EPW,), jnp.int32),
    ] + [pltpu.VMEM((GCH, D), jnp.float32)] * 6
      + [pltpu.SemaphoreType.DMA] * 12,
)


def _extract48(ref, pos):
    total = jnp.float32(0)
    for j in range(3):
        v = ref[pl.ds(j * 16, 16)]
        lane = lax.broadcasted_iota(jnp.int32, (16,), 0) + (j * 16)
        total = total + jnp.sum(jnp.where(lane == pos, v, 0.0))
    return total.astype(jnp.int32)


def _soft_body(lg_hbm, vf_hbm, di_hbm, eb_hbm, msg_hbm,
               acc, msum, ssum, lbuf, dbuf, vbuf, abuf, ebv, sem):
    wid = lax.axis_index("c") * 16 + lax.axis_index("s")
    n0 = wid * NPW
    iota = lax.broadcasted_iota(jnp.int32, (16,), 0)
    half = lax.shift_right_logical(iota, 3)
    lane7 = jnp.bitwise_and(iota, 7)
    n0f = jnp.float32(n0)
    pltpu.sync_copy(eb_hbm, ebv)
    e0 = _extract48(ebv, wid)
    e1 = _extract48(ebv, wid + 1)
    estart = (e0 // 8) * 8
    nch = (e1 - estart + CE - 1) // CE
    nch3 = (e1 - estart + CE3 - 1) // CE3

    zz = jnp.zeros((16,), jnp.float32)

    def _zacc(j, c):
        acc[pl.ds(j * 16, 16)] = zz
        return c
    lax.fori_loop(0, ACCW // 16, _zacc, 0)

    def _zs(j, c):
        ssum[pl.ds(j * 16, 16)] = zz
        return c
    lax.fori_loop(0, MSW // 16, _zs, 0)

    def _edge_vec(base, r):
        lg = lbuf[pl.ds(r * 16, 16)]
        eg = base + 2 * r + half
        mk = (eg >= e0) & (eg < e1)
        dv = plsc.load_gather(dbuf, [2 * r + half], mask=mk)
        dl = jnp.clip(dv - n0f, 0.0, float(NPW - 1))
        fidx = (dl * 8.0).astype(jnp.int32) + lane7
        return lg, fidx, mk

    def p1_chunk(c, carry):
        base = estart + c * CE
        pltpu.sync_copy(lg_hbm.at[pl.ds(base * 8, CE * 8)], lbuf)
        pltpu.sync_copy(di_hbm.at[pl.ds(base, CE)], dbuf)

        def it(r, cc):
            lg, fidx, mk = _edge_vec(base, r)
            plsc.store_scatter(msum, [fidx], lg, mask=mk)
            return cc
        lax.fori_loop(0, CE // 2, it, 0)
        return carry
    lax.fori_loop(0, nch, p1_chunk, 0)

    def p2_chunk(c, carry):
        base = estart + c * CE
        pltpu.sync_copy(lg_hbm.at[pl.ds(base * 8, CE * 8)], lbuf)
        pltpu.sync_copy(di_hbm.at[pl.ds(base, CE)], dbuf)

        def it(r, cc):
            lg, fidx, mk = _edge_vec(base, r)
            m = plsc.load_gather(msum, [fidx], mask=mk)
            ex = jnp.exp(lg - m)
            plsc.addupdate_scatter(ssum, [fidx], ex, mask=mk & (iota < 8))
            plsc.addupdate_scatter(ssum, [fidx], ex, mask=mk & (iota >= 8))
            return cc
        lax.fori_loop(0, CE // 2, it, 0)
        return carry
    lax.fori_loop(0, nch, p2_chunk, 0)

    def p3_chunk(c, carry):
        base = estart + c * CE3
        pltpu.sync_copy(lg_hbm.at[pl.ds(base * 8, CE3 * 8)], lbuf.at[pl.ds(0, CE3 * 8)])
        pltpu.sync_copy(di_hbm.at[pl.ds(base, CE3)], dbuf.at[pl.ds(0, CE3)])
        pltpu.sync_copy(vf_hbm.at[pl.ds(base * D, CE3 * D)], vbuf)

        def ita(r, cc):
            lg, fidx, mk = _edge_vec(base, r)
            m = plsc.load_gather(msum, [fidx], mask=mk)
            s = plsc.load_gather(ssum, [fidx], mask=mk)
            abuf[pl.ds(r * 16, 16)] = jnp.exp(lg - m) / (s + 1e-16)
            return cc
        lax.fori_loop(0, CE3 * 8 // 16, ita, 0)

        def ite(e, cc):
            eg = jnp.full((16,), base + e, jnp.int32)
            mk = (eg >= e0) & (eg < e1)
            dv = plsc.load_gather(dbuf, [jnp.full((16,), e, jnp.int32)], mask=mk)
            bl = (jnp.clip(dv - n0f, 0.0, float(NPW - 1)) * float(D)).astype(jnp.int32) + iota
            for hh in range(H):
                av = plsc.load_gather(abuf, [jnp.full((16,), e * 8 + hh, jnp.int32)], mask=mk)
                for f in (2 * hh, 2 * hh + 1):
                    vv = vbuf[pl.ds(e * D + f * 16, 16)]
                    plsc.addupdate_scatter(acc, [bl + f * 16], vv * av, mask=mk)
            return cc
        lax.fori_loop(0, CE3, ite, 0)
        return carry
    lax.fori_loop(0, nch3, p3_chunk, 0)

    pltpu.sync_copy(acc, msg_hbm.at[pl.ds(n0 * D, NPW * D)])


_soft = pl.kernel(
    _soft_body,
    out_type=jax.ShapeDtypeStruct((NPAD * D,), jnp.float32),
    mesh=_mesh,
    compiler_params=pltpu.CompilerParams(needs_layout_passes=False),
    scratch_types=[
        pltpu.VMEM((ACCW,), jnp.float32),
        pltpu.VMEM((MSW,), jnp.float32),
        pltpu.VMEM((MSW,), jnp.float32),
        pltpu.VMEM((CE * 8,), jnp.float32),
        pltpu.VMEM((CE,), jnp.float32),
        pltpu.VMEM((CE3 * D,), jnp.float32),
        pltpu.VMEM((CE3 * 8,), jnp.float32),
        pltpu.VMEM((48,), jnp.float32),
        pltpu.SemaphoreType.DMA,
    ],
)


def _readout_body(g_ref, w1_ref, b1_ref, w2_ref, b2_ref, w3_ref, b3_ref, out_ref):
    g = g_ref[...]
    g = jax.nn.relu(jnp.dot(g, w1_ref[...], preferred_element_type=jnp.float32) + b1_ref[...])
    g = jax.nn.relu(jnp.dot(g, w2_ref[...], preferred_element_type=jnp.float32) + b2_ref[...])
    out_ref[...] = jnp.dot(g, w3_ref[...], preferred_element_type=jnp.float32) + b3_ref[...]


def kernel(x, edge_index, edge_attr, pos, batch, params):
    src = edge_index[0].astype(jnp.int32)
    dst = edge_index[1].astype(jnp.int32)
    perm = jnp.argsort(dst)
    dst_s = dst[perm]
    src_s = src[perm]
    ea_s = edge_attr[perm]
    ebounds = jnp.searchsorted(dst_s, jnp.arange(0, NPAD + NPW, NPW, dtype=jnp.int32)
                               ).astype(jnp.int32)
    ebounds = jnp.pad(ebounds, (0, 48 - NW - 1), constant_values=E)
    dst_s = jnp.pad(dst_s, (0, EPAD - E))
    src_s = jnp.pad(src_s, (0, EPAD - E))
    ea_s = jnp.pad(ea_s, ((0, EPAD - E), (0, 0)))
    dst_sf = dst_s.astype(jnp.float32)
    ebounds_f = ebounds.astype(jnp.float32)

    h = x @ params['node_emb_W'] + params['node_emb_b']
    freq = jnp.linspace(0.0, 10.0, NFREQ, dtype=jnp.float32)
    enc = []
    for i in range(3):
        ang = pos[:, i:i + 1] * freq[None, :]
        enc.append(jnp.sin(ang))
        enc.append(jnp.cos(ang))
    pe = jnp.concatenate(enc, axis=1) @ params['pos_W'] + params['pos_b']
    h = h + pe

    EW = params['edge_emb_W']
    ebias = params['edge_emb_b']
    scale = 1.0 / (float(C) ** 0.5)
    for l in range(L):
        p = params['layers'][l]
        F = EW @ p['We']
        g = ebias @ p['We'] + p['be']
        res = h
        Q = h @ p['Wq'] + p['bq']
        K = h @ p['Wk'] + (p['bk'] + g)
        V = h @ p['Wv'] + (p['bv'] + g)
        qr, kr, vr = _gather(Q, K, V, dst_s, src_s)
        u = ea_s @ F
        kf = kr + u
        vf = vr + u
        logits = (qr * kf).reshape(EPAD, H, C).sum(-1) * scale
        msg = _soft(logits.reshape(-1), vf.reshape(-1), dst_sf, ebounds_f)
        msg = msg.reshape(NPAD, D)[:N]
        r = h @ p['Wskip'] + p['bskip']
        beta = jax.nn.sigmoid(jnp.concatenate([msg, r, msg - r], axis=1) @ p['Wbeta'] + p['bbeta'])
        out = beta * r + (1.0 - beta) * msg
        mu = out.mean(-1, keepdims=True)
        var = out.var(-1, keepdims=True)
        out = (out - mu) / jnp.sqrt(var + 1e-5) * p['ln_g'] + p['ln_b']
        out = jax.nn.relu(out)
        if l > 0:
            out = out + res
        h = out

    add = jax.ops.segment_sum(h, batch, num_segments=G)
    cnt = jax.ops.segment_sum(jnp.ones((N, 1), jnp.float32), batch, num_segments=G)
    mean = add / jnp.maximum(cnt, 1.0)
    mx = jax.ops.segment_max(h, batch, num_segments=G)
    mx = jnp.where(jnp.isfinite(mx), mx, 0.0)
    gfeat = jnp.concatenate([mean, mx, add], axis=1)
    out = pl.pallas_call(
        _readout_body,
        out_shape=jax.ShapeDtypeStruct((G, OUT), jnp.float32),
    )(gfeat, params['r_W1'], params['r_b1'], params['r_W2'], params['r_b2'],
      params['r_W3'], params['r_b3'])
    return out


# speedup vs baseline: 1.8747x; 1.1534x over previous
import functools

import jax
import jax.numpy as jnp
from jax import lax
from jax.experimental import pallas as pl
from jax.experimental.pallas import tpu as pltpu
from jax.experimental.pallas import tpu_sc as plsc

N = 10000
E = 160000
D = 256
H = 8
C = D // H
L = 6
ED = 4
OUT = 128
G = 64
NFREQ = D // 6

NW = 32
GCH = 64
EPAD = 163840
EPW = EPAD // NW
GITER = EPW // (2 * GCH)

NPW = 313
NPAD = NW * NPW
CE = 512
CE3 = 128
ACCW = NPW * D
MSW = 2512

_mesh = plsc.VectorSubcoreMesh(core_axis_name="c", subcore_axis_name="s")


def _gather_body(q_hbm, k_hbm, v_hbm, di_hbm, si_hbm, qo, ko, vo,
                 idxd, idxs, qba, kba, vba, qbb, kbb, vbb,
                 sqa, ska, sva, sqb, skb, svb,
                 wqa, wka, wva, wqb, wkb, wvb):
    wid = lax.axis_index("c") * 16 + lax.axis_index("s")
    wbase = wid * EPW
    pltpu.sync_copy(di_hbm.at[pl.ds(wbase, EPW)], idxd)
    pltpu.sync_copy(si_hbm.at[pl.ds(wbase, EPW)], idxs)

    def it(i, carry):
        oa = 2 * i * GCH
        ob = oa + GCH
        ga = [
            pltpu.async_copy(q_hbm.at[idxd.at[pl.ds(oa, GCH)]], qba, sqa),
            pltpu.async_copy(k_hbm.at[idxs.at[pl.ds(oa, GCH)]], kba, ska),
            pltpu.async_copy(v_hbm.at[idxs.at[pl.ds(oa, GCH)]], vba, sva),
        ]
        gb = [
            pltpu.async_copy(q_hbm.at[idxd.at[pl.ds(ob, GCH)]], qbb, sqb),
            pltpu.async_copy(k_hbm.at[idxs.at[pl.ds(ob, GCH)]], kbb, skb),
            pltpu.async_copy(v_hbm.at[idxs.at[pl.ds(ob, GCH)]], vbb, svb),
        ]
        for g in ga:
            g.wait()
        wa = [
            pltpu.async_copy(qba, qo.at[pl.ds(wbase + oa, GCH)], wqa),
            pltpu.async_copy(kba, ko.at[pl.ds(wbase + oa, GCH)], wka),
            pltpu.async_copy(vba, vo.at[pl.ds(wbase + oa, GCH)], wva),
        ]
        for g in gb:
            g.wait()
        wb = [
            pltpu.async_copy(qbb, qo.at[pl.ds(wbase + ob, GCH)], wqb),
            pltpu.async_copy(kbb, ko.at[pl.ds(wbase + ob, GCH)], wkb),
            pltpu.async_copy(vbb, vo.at[pl.ds(wbase + ob, GCH)], wvb),
        ]
        for w in wa + wb:
            w.wait()
        return carry

    lax.fori_loop(0, GITER, it, 0)


_gather = pl.kernel(
    _gather_body,
    out_type=[jax.ShapeDtypeStruct((EPAD, D), jnp.float32)] * 3,
    mesh=_mesh,
    scratch_types=[
        pltpu.VMEM((EPW,), jnp.int32),
        pltpu.VMEM((EPW,), jnp.int32),
    ] + [pltpu.VMEM((GCH, D), jnp.float32)] * 6
      + [pltpu.SemaphoreType.DMA] * 12,
)


def _extract48(ref, pos):
    total = jnp.float32(0)
    for j in range(3):
        v = ref[pl.ds(j * 16, 16)]
        lane = lax.broadcasted_iota(jnp.int32, (16,), 0) + (j * 16)
        total = total + jnp.sum(jnp.where(lane == pos, v, 0.0))
    return total.astype(jnp.int32)


def _soft_body(lg_hbm, vf_hbm, di_hbm, eb_hbm, msg_hbm,
               acc, msum, ssum, lbuf, dbuf, vbuf, abuf, ebv, sem):
    wid = lax.axis_index("c") * 16 + lax.axis_index("s")
    n0 = wid * NPW
    iota = lax.broadcasted_iota(jnp.int32, (16,), 0)
    half = lax.shift_right_logical(iota, 3)
    lane7 = jnp.bitwise_and(iota, 7)
    n0f = jnp.float32(n0)
    pltpu.sync_copy(eb_hbm, ebv)
    e0 = _extract48(ebv, wid)
    e1 = _extract48(ebv, wid + 1)
    estart = (e0 // 8) * 8
    nch = (e1 - estart + CE - 1) // CE
    nch3 = (e1 - estart + CE3 - 1) // CE3

    zz = jnp.zeros((16,), jnp.float32)

    def _zacc(j, c):
        acc[pl.ds(j * 16, 16)] = zz
        return c
    lax.fori_loop(0, ACCW // 16, _zacc, 0)

    def _zs(j, c):
        ssum[pl.ds(j * 16, 16)] = zz
        return c
    lax.fori_loop(0, MSW // 16, _zs, 0)

    def _edge_vec(base, r):
        lg = lbuf[pl.ds(r * 16, 16)]
        eg = base + 2 * r + half
        mk = (eg >= e0) & (eg < e1)
        dv = plsc.load_gather(dbuf, [2 * r + half], mask=mk)
        dl = jnp.clip(dv - n0f, 0.0, float(NPW - 1))
        fidx = (dl * 8.0).astype(jnp.int32) + lane7
        return lg, fidx, mk

    def p1_chunk(c, carry):
        base = estart + c * CE
        pltpu.sync_copy(lg_hbm.at[pl.ds(base * 8, CE * 8)], lbuf)
        pltpu.sync_copy(di_hbm.at[pl.ds(base, CE)], dbuf)

        def it(r, cc):
            lg, fidx, mk = _edge_vec(base, r)
            plsc.store_scatter(msum, [fidx], lg, mask=mk)
            return cc
        lax.fori_loop(0, CE // 2, it, 0)
        return carry
    lax.fori_loop(0, nch, p1_chunk, 0)

    def p2_chunk(c, carry):
        base = estart + c * CE
        pltpu.sync_copy(lg_hbm.at[pl.ds(base * 8, CE * 8)], lbuf)
        pltpu.sync_copy(di_hbm.at[pl.ds(base, CE)], dbuf)

        def it(r, cc):
            lg, fidx, mk = _edge_vec(base, r)
            m = plsc.load_gather(msum, [fidx], mask=mk)
            ex = jnp.exp(lg - m)
            plsc.addupdate_scatter(ssum, [fidx], ex, mask=mk & (iota < 8))
            plsc.addupdate_scatter(ssum, [fidx], ex, mask=mk & (iota >= 8))
            return cc
        lax.fori_loop(0, CE // 2, it, 0)
        return carry
    lax.fori_loop(0, nch, p2_chunk, 0)

    def p3_chunk(c, carry):
        base = estart + c * CE3
        pltpu.sync_copy(lg_hbm.at[pl.ds(base * 8, CE3 * 8)], lbuf.at[pl.ds(0, CE3 * 8)])
        pltpu.sync_copy(di_hbm.at[pl.ds(base, CE3)], dbuf.at[pl.ds(0, CE3)])
        pltpu.sync_copy(vf_hbm.at[pl.ds(base * D, CE3 * D)], vbuf)

        def ita(r, cc):
            lg, fidx, mk = _edge_vec(base, r)
            m = plsc.load_gather(msum, [fidx], mask=mk)
            s = plsc.load_gather(ssum, [fidx], mask=mk)
            abuf[pl.ds(r * 16, 16)] = jnp.exp(lg - m) / (s + 1e-16)
            return cc
        lax.fori_loop(0, CE3 * 8 // 16, ita, 0)

        def ite(e, cc):
            eg = jnp.full((16,), base + e, jnp.int32)
            mk = (eg >= e0) & (eg < e1)
            dv = plsc.load_gather(dbuf, [jnp.full((16,), e, jnp.int32)], mask=mk)
            bl = (jnp.clip(dv - n0f, 0.0, float(NPW - 1)) * float(D)).astype(jnp.int32) + iota
            for hh in range(H):
                av = plsc.load_gather(abuf, [jnp.full((16,), e * 8 + hh, jnp.int32)], mask=mk)
                for f in (2 * hh, 2 * hh + 1):
                    vv = vbuf[pl.ds(e * D + f * 16, 16)]
                    plsc.addupdate_scatter(acc, [bl + f * 16], vv * av, mask=mk)
            return cc
        lax.fori_loop(0, CE3, ite, 0)
        return carry
    lax.fori_loop(0, nch3, p3_chunk, 0)

    pltpu.sync_copy(acc, msg_hbm.at[pl.ds(n0 * D, NPW * D)])


_soft = pl.kernel(
    _soft_body,
    out_type=jax.ShapeDtypeStruct((NPAD * D,), jnp.float32),
    mesh=_mesh,
    compiler_params=pltpu.CompilerParams(needs_layout_passes=False),
    scratch_types=[
        pltpu.VMEM((ACCW,), jnp.float32),
        pltpu.VMEM((MSW,), jnp.float32),
        pltpu.VMEM((MSW,), jnp.float32),
        pltpu.VMEM((CE * 8,), jnp.float32),
        pltpu.VMEM((CE,), jnp.float32),
        pltpu.VMEM((CE3 * D,), jnp.float32),
        pltpu.VMEM((CE3 * 8,), jnp.float32),
        pltpu.VMEM((48,), jnp.float32),
        pltpu.SemaphoreType.DMA,
    ],
)


BE = 2048
BN = 400


def _edge_body(qr_ref, kr_ref, vr_ref, ea_ref, f_ref, sel_ref, lg_ref, vf_ref):
    u = jnp.dot(ea_ref[...], f_ref[...], preferred_element_type=jnp.float32)
    kf = kr_ref[...] + u
    lg_ref[...] = jnp.dot(qr_ref[...] * kf, sel_ref[...],
                          preferred_element_type=jnp.float32)
    vf_ref[...] = vr_ref[...] + u


_edge_tc = pl.pallas_call(
    _edge_body,
    grid=(EPAD // BE,),
    in_specs=[
        pl.BlockSpec((BE, D), lambda i: (i, 0)),
        pl.BlockSpec((BE, D), lambda i: (i, 0)),
        pl.BlockSpec((BE, D), lambda i: (i, 0)),
        pl.BlockSpec((BE, ED), lambda i: (i, 0)),
        pl.BlockSpec((ED, D), lambda i: (0, 0)),
        pl.BlockSpec((D, H), lambda i: (0, 0)),
    ],
    out_specs=[
        pl.BlockSpec((BE, H), lambda i: (i, 0)),
        pl.BlockSpec((BE, D), lambda i: (i, 0)),
    ],
    out_shape=[
        jax.ShapeDtypeStruct((EPAD, H), jnp.float32),
        jax.ShapeDtypeStruct((EPAD, D), jnp.float32),
    ],
)


def _qkvr_body(h_ref, w_ref, b_ref, o_ref):
    o_ref[...] = jnp.dot(h_ref[...], w_ref[...],
                         preferred_element_type=jnp.float32) + b_ref[...]


_qkvr = pl.pallas_call(
    _qkvr_body,
    grid=(N // BN,),
    in_specs=[
        pl.BlockSpec((BN, D), lambda i: (i, 0)),
        pl.BlockSpec((D, 4 * D), lambda i: (0, 0)),
        pl.BlockSpec((1, 4 * D), lambda i: (0, 0)),
    ],
    out_specs=pl.BlockSpec((BN, 4 * D), lambda i: (i, 0)),
    out_shape=jax.ShapeDtypeStruct((N, 4 * D), jnp.float32),
)


def _readout_body(g_ref, w1_ref, b1_ref, w2_ref, b2_ref, w3_ref, b3_ref, out_ref):
    g = g_ref[...]
    g = jax.nn.relu(jnp.dot(g, w1_ref[...], preferred_element_type=jnp.float32) + b1_ref[...])
    g = jax.nn.relu(jnp.dot(g, w2_ref[...], preferred_element_type=jnp.float32) + b2_ref[...])
    out_ref[...] = jnp.dot(g, w3_ref[...], preferred_element_type=jnp.float32) + b3_ref[...]


def kernel(x, edge_index, edge_attr, pos, batch, params):
    src = edge_index[0].astype(jnp.int32)
    dst = edge_index[1].astype(jnp.int32)
    perm = jnp.argsort(dst)
    dst_s = dst[perm]
    src_s = src[perm]
    ea_s = edge_attr[perm]
    ebounds = jnp.searchsorted(dst_s, jnp.arange(0, NPAD + NPW, NPW, dtype=jnp.int32)
                               ).astype(jnp.int32)
    ebounds = jnp.pad(ebounds, (0, 48 - NW - 1), constant_values=E)
    dst_s = jnp.pad(dst_s, (0, EPAD - E))
    src_s = jnp.pad(src_s, (0, EPAD - E))
    ea_s = jnp.pad(ea_s, ((0, EPAD - E), (0, 0)))
    dst_sf = dst_s.astype(jnp.float32)
    ebounds_f = ebounds.astype(jnp.float32)

    h = x @ params['node_emb_W'] + params['node_emb_b']
    freq = jnp.linspace(0.0, 10.0, NFREQ, dtype=jnp.float32)
    enc = []
    for i in range(3):
        ang = pos[:, i:i + 1] * freq[None, :]
        enc.append(jnp.sin(ang))
        enc.append(jnp.cos(ang))
    pe = jnp.concatenate(enc, axis=1) @ params['pos_W'] + params['pos_b']
    h = h + pe

    EW = params['edge_emb_W']
    ebias = params['edge_emb_b']
    scale = 1.0 / (float(C) ** 0.5)
    sel = jnp.kron(jnp.eye(H, dtype=jnp.float32),
                   jnp.full((C, 1), scale, jnp.float32))
    for l in range(L):
        p = params['layers'][l]
        F = EW @ p['We']
        g = ebias @ p['We'] + p['be']
        res = h
        w4 = jnp.concatenate([p['Wq'], p['Wk'], p['Wv'], p['Wskip']], axis=1)
        b4 = jnp.concatenate([p['bq'], p['bk'] + g, p['bv'] + g, p['bskip']])
        x4 = _qkvr(h, w4, b4.reshape(1, 4 * D))
        Q = x4[:, :D]
        K = x4[:, D:2 * D]
        V = x4[:, 2 * D:3 * D]
        r = x4[:, 3 * D:]
        qr, kr, vr = _gather(Q, K, V, dst_s, src_s)
        logits, vf = _edge_tc(qr, kr, vr, ea_s, F, sel)
        msg = _soft(logits.reshape(-1), vf.reshape(-1), dst_sf, ebounds_f)
        msg = msg.reshape(NPAD, D)[:N]
        wb = p['Wbeta'][:, 0]
        wbm = wb[:D] + wb[2 * D:]
        wbr = wb[D:2 * D] - wb[2 * D:]
        beta = jax.nn.sigmoid((msg @ wbm + r @ wbr + p['bbeta'])[:, None])
        out = beta * r + (1.0 - beta) * msg
        mu = out.mean(-1, keepdims=True)
        var = out.var(-1, keepdims=True)
        out = (out - mu) / jnp.sqrt(var + 1e-5) * p['ln_g'] + p['ln_b']
        out = jax.nn.relu(out)
        if l > 0:
            out = out + res
        h = out

    add = jax.ops.segment_sum(h, batch, num_segments=G)
    cnt = jax.ops.segment_sum(jnp.ones((N, 1), jnp.float32), batch, num_segments=G)
    mean = add / jnp.maximum(cnt, 1.0)
    mx = jax.ops.segment_max(h, batch, num_segments=G)
    mx = jnp.where(jnp.isfinite(mx), mx, 0.0)
    gfeat = jnp.concatenate([mean, mx, add], axis=1)
    out = pl.pallas_call(
        _readout_body,
        out_shape=jax.ShapeDtypeStruct((G, OUT), jnp.float32),
    )(gfeat, params['r_W1'], params['r_b1'], params['r_W2'], params['r_b2'],
      params['r_W3'], params['r_b3'])
    return out


# speedup vs baseline: 1.9128x; 1.0203x over previous
import functools

import jax
import jax.numpy as jnp
from jax import lax
from jax.experimental import pallas as pl
from jax.experimental.pallas import tpu as pltpu
from jax.experimental.pallas import tpu_sc as plsc

N = 10000
E = 160000
D = 256
H = 8
C = D // H
L = 6
ED = 4
OUT = 128
G = 64
NFREQ = D // 6

NW = 32
GCH = 64
EPAD = 163840
EPW = EPAD // NW
GITER = EPW // (2 * GCH)

NPW = 313
NPAD = NW * NPW
CE = 512
CE3 = 64
ACCW = NPW * D
MSW = 2512

_mesh = plsc.VectorSubcoreMesh(core_axis_name="c", subcore_axis_name="s")


def _gather_body(q_hbm, k_hbm, v_hbm, di_hbm, si_hbm, qo, ko, vo,
                 idxd, idxs, qba, kba, vba, qbb, kbb, vbb,
                 sqa, ska, sva, sqb, skb, svb,
                 wqa, wka, wva, wqb, wkb, wvb):
    wid = lax.axis_index("c") * 16 + lax.axis_index("s")
    wbase = wid * EPW
    pltpu.sync_copy(di_hbm.at[pl.ds(wbase, EPW)], idxd)
    pltpu.sync_copy(si_hbm.at[pl.ds(wbase, EPW)], idxs)

    def it(i, carry):
        oa = 2 * i * GCH
        ob = oa + GCH
        ga = [
            pltpu.async_copy(q_hbm.at[idxd.at[pl.ds(oa, GCH)]], qba, sqa),
            pltpu.async_copy(k_hbm.at[idxs.at[pl.ds(oa, GCH)]], kba, ska),
            pltpu.async_copy(v_hbm.at[idxs.at[pl.ds(oa, GCH)]], vba, sva),
        ]
        gb = [
            pltpu.async_copy(q_hbm.at[idxd.at[pl.ds(ob, GCH)]], qbb, sqb),
            pltpu.async_copy(k_hbm.at[idxs.at[pl.ds(ob, GCH)]], kbb, skb),
            pltpu.async_copy(v_hbm.at[idxs.at[pl.ds(ob, GCH)]], vbb, svb),
        ]
        for g in ga:
            g.wait()
        wa = [
            pltpu.async_copy(qba, qo.at[pl.ds(wbase + oa, GCH)], wqa),
            pltpu.async_copy(kba, ko.at[pl.ds(wbase + oa, GCH)], wka),
            pltpu.async_copy(vba, vo.at[pl.ds(wbase + oa, GCH)], wva),
        ]
        for g in gb:
            g.wait()
        wb = [
            pltpu.async_copy(qbb, qo.at[pl.ds(wbase + ob, GCH)], wqb),
            pltpu.async_copy(kbb, ko.at[pl.ds(wbase + ob, GCH)], wkb),
            pltpu.async_copy(vbb, vo.at[pl.ds(wbase + ob, GCH)], wvb),
        ]
        for w in wa + wb:
            w.wait()
        return carry

    lax.fori_loop(0, GITER, it, 0)


_gather = pl.kernel(
    _gather_body,
    out_type=[jax.ShapeDtypeStruct((EPAD, D), jnp.float32)] * 3,
    mesh=_mesh,
    scratch_types=[
        pltpu.VMEM((EPW,), jnp.int32),
        pltpu.VMEM((EPW,), jnp.int32),
    ] + [pltpu.VMEM((GCH, D), jnp.float32)] * 6
      + [pltpu.SemaphoreType.DMA] * 12,
)


def _extract48(ref, pos):
    total = jnp.float32(0)
    for j in range(3):
        v = ref[pl.ds(j * 16, 16)]
        lane = lax.broadcasted_iota(jnp.int32, (16,), 0) + (j * 16)
        total = total + jnp.sum(jnp.where(lane == pos, v, 0.0))
    return total.astype(jnp.int32)


def _soft_body(lg_hbm, vf_hbm, di_hbm, eb_hbm, msg_hbm,
               acc, msum, ssum, lbuf, dbuf, ebv,
               vbufa, lbufa, dbufa, abufa, vbufb, lbufb, dbufb, abufb,
               sva3, sla3, sda3, svb3, slb3, sdb3):
    wid = lax.axis_index("c") * 16 + lax.axis_index("s")
    n0 = wid * NPW
    iota = lax.broadcasted_iota(jnp.int32, (16,), 0)
    half = lax.shift_right_logical(iota, 3)
    lane7 = jnp.bitwise_and(iota, 7)
    n0f = jnp.float32(n0)
    pltpu.sync_copy(eb_hbm, ebv)
    e0 = _extract48(ebv, wid)
    e1 = _extract48(ebv, wid + 1)
    estart = (e0 // 8) * 8
    nch = (e1 - estart + CE - 1) // CE
    nch3 = (e1 - estart + CE3 - 1) // CE3

    zz = jnp.zeros((16,), jnp.float32)

    def _zacc(j, c):
        acc[pl.ds(j * 16, 16)] = zz
        return c
    lax.fori_loop(0, ACCW // 16, _zacc, 0)

    def _zs(j, c):
        ssum[pl.ds(j * 16, 16)] = zz
        return c
    lax.fori_loop(0, MSW // 16, _zs, 0)

    def _edge_vec(base, r):
        lg = lbuf[pl.ds(r * 16, 16)]
        eg = base + 2 * r + half
        mk = (eg >= e0) & (eg < e1)
        dv = plsc.load_gather(dbuf, [2 * r + half], mask=mk)
        dl = jnp.clip(dv - n0f, 0.0, float(NPW - 1))
        fidx = (dl * 8.0).astype(jnp.int32) + lane7
        return lg, fidx, mk

    def p1_chunk(c, carry):
        base = estart + c * CE
        pltpu.sync_copy(lg_hbm.at[pl.ds(base * 8, CE * 8)], lbuf)
        pltpu.sync_copy(di_hbm.at[pl.ds(base, CE)], dbuf)

        def it(r, cc):
            lg, fidx, mk = _edge_vec(base, r)
            plsc.store_scatter(msum, [fidx], lg, mask=mk)
            return cc
        lax.fori_loop(0, CE // 2, it, 0)
        return carry
    lax.fori_loop(0, nch, p1_chunk, 0)

    def p2_chunk(c, carry):
        base = estart + c * CE
        pltpu.sync_copy(lg_hbm.at[pl.ds(base * 8, CE * 8)], lbuf)
        pltpu.sync_copy(di_hbm.at[pl.ds(base, CE)], dbuf)

        def it(r, cc):
            lg, fidx, mk = _edge_vec(base, r)
            m = plsc.load_gather(msum, [fidx], mask=mk)
            ex = jnp.exp(lg - m)
            plsc.addupdate_scatter(ssum, [fidx], ex, mask=mk & (iota < 8))
            plsc.addupdate_scatter(ssum, [fidx], ex, mask=mk & (iota >= 8))
            return cc
        lax.fori_loop(0, CE // 2, it, 0)
        return carry
    lax.fori_loop(0, nch, p2_chunk, 0)

    def _proc3(base, vbuf3, lbuf3, dbuf3, abuf3):
        def ita(r, cc):
            lg = lbuf3[pl.ds(r * 16, 16)]
            eg = base + 2 * r + half
            mk = (eg >= e0) & (eg < e1)
            dv = plsc.load_gather(dbuf3, [2 * r + half], mask=mk)
            dl = jnp.clip(dv - n0f, 0.0, float(NPW - 1))
            fidx = (dl * 8.0).astype(jnp.int32) + lane7
            m = plsc.load_gather(msum, [fidx], mask=mk)
            s = plsc.load_gather(ssum, [fidx], mask=mk)
            abuf3[pl.ds(r * 16, 16)] = jnp.exp(lg - m) / (s + 1e-16)
            return cc
        lax.fori_loop(0, CE3 * 8 // 16, ita, 0)

        def ite(e, cc):
            eg = jnp.full((16,), base + e, jnp.int32)
            mk = (eg >= e0) & (eg < e1)
            dv = plsc.load_gather(dbuf3, [jnp.full((16,), e, jnp.int32)], mask=mk)
            bl = (jnp.clip(dv - n0f, 0.0, float(NPW - 1)) * float(D)).astype(jnp.int32) + iota
            for hh in range(H):
                av = plsc.load_gather(abuf3, [jnp.full((16,), e * 8 + hh, jnp.int32)], mask=mk)
                for f in (2 * hh, 2 * hh + 1):
                    vv = vbuf3[pl.ds(e * D + f * 16, 16)]
                    plsc.addupdate_scatter(acc, [bl + f * 16], vv * av, mask=mk)
            return cc
        lax.fori_loop(0, CE3, ite, 0)

    npair = (e1 - estart + 2 * CE3 - 1) // (2 * CE3)

    def p3_pair(j, carry):
        ba = estart + j * 2 * CE3
        bb = ba + CE3
        da = [
            pltpu.async_copy(vf_hbm.at[pl.ds(ba * D, CE3 * D)], vbufa, sva3),
            pltpu.async_copy(lg_hbm.at[pl.ds(ba * 8, CE3 * 8)], lbufa, sla3),
            pltpu.async_copy(di_hbm.at[pl.ds(ba, CE3)], dbufa, sda3),
        ]
        db = [
            pltpu.async_copy(vf_hbm.at[pl.ds(bb * D, CE3 * D)], vbufb, svb3),
            pltpu.async_copy(lg_hbm.at[pl.ds(bb * 8, CE3 * 8)], lbufb, slb3),
            pltpu.async_copy(di_hbm.at[pl.ds(bb, CE3)], dbufb, sdb3),
        ]
        for d in da:
            d.wait()
        _proc3(ba, vbufa, lbufa, dbufa, abufa)
        for d in db:
            d.wait()
        _proc3(bb, vbufb, lbufb, dbufb, abufb)
        return carry
    lax.fori_loop(0, npair, p3_pair, 0)

    pltpu.sync_copy(acc, msg_hbm.at[pl.ds(n0 * D, NPW * D)])


_soft = pl.kernel(
    _soft_body,
    out_type=jax.ShapeDtypeStruct((NPAD * D,), jnp.float32),
    mesh=_mesh,
    compiler_params=pltpu.CompilerParams(needs_layout_passes=False),
    scratch_types=[
        pltpu.VMEM((ACCW,), jnp.float32),
        pltpu.VMEM((MSW,), jnp.float32),
        pltpu.VMEM((MSW,), jnp.float32),
        pltpu.VMEM((CE * 8,), jnp.float32),
        pltpu.VMEM((CE,), jnp.float32),
        pltpu.VMEM((48,), jnp.float32),
        pltpu.VMEM((CE3 * D,), jnp.float32),
        pltpu.VMEM((CE3 * 8,), jnp.float32),
        pltpu.VMEM((CE3,), jnp.float32),
        pltpu.VMEM((CE3 * 8,), jnp.float32),
        pltpu.VMEM((CE3 * D,), jnp.float32),
        pltpu.VMEM((CE3 * 8,), jnp.float32),
        pltpu.VMEM((CE3,), jnp.float32),
        pltpu.VMEM((CE3 * 8,), jnp.float32),
    ] + [pltpu.SemaphoreType.DMA] * 6,
)


BE = 2048
BN = 400


def _edge_body(qr_ref, kr_ref, vr_ref, ea_ref, f_ref, sel_ref, lg_ref, vf_ref):
    u = jnp.dot(ea_ref[...], f_ref[...], preferred_element_type=jnp.float32)
    kf = kr_ref[...] + u
    lg_ref[...] = jnp.dot(qr_ref[...] * kf, sel_ref[...],
                          preferred_element_type=jnp.float32)
    vf_ref[...] = vr_ref[...] + u


_edge_tc = pl.pallas_call(
    _edge_body,
    grid=(EPAD // BE,),
    in_specs=[
        pl.BlockSpec((BE, D), lambda i: (i, 0)),
        pl.BlockSpec((BE, D), lambda i: (i, 0)),
        pl.BlockSpec((BE, D), lambda i: (i, 0)),
        pl.BlockSpec((BE, ED), lambda i: (i, 0)),
        pl.BlockSpec((ED, D), lambda i: (0, 0)),
        pl.BlockSpec((D, H), lambda i: (0, 0)),
    ],
    out_specs=[
        pl.BlockSpec((BE, H), lambda i: (i, 0)),
        pl.BlockSpec((BE, D), lambda i: (i, 0)),
    ],
    out_shape=[
        jax.ShapeDtypeStruct((EPAD, H), jnp.float32),
        jax.ShapeDtypeStruct((EPAD, D), jnp.float32),
    ],
)


def _qkvr_body(h_ref, w_ref, b_ref, o_ref):
    o_ref[...] = jnp.dot(h_ref[...], w_ref[...],
                         preferred_element_type=jnp.float32) + b_ref[...]


_qkvr = pl.pallas_call(
    _qkvr_body,
    grid=(N // BN,),
    in_specs=[
        pl.BlockSpec((BN, D), lambda i: (i, 0)),
        pl.BlockSpec((D, 4 * D), lambda i: (0, 0)),
        pl.BlockSpec((1, 4 * D), lambda i: (0, 0)),
    ],
    out_specs=pl.BlockSpec((BN, 4 * D), lambda i: (i, 0)),
    out_shape=jax.ShapeDtypeStruct((N, 4 * D), jnp.float32),
)


def _readout_body(g_ref, w1_ref, b1_ref, w2_ref, b2_ref, w3_ref, b3_ref, out_ref):
    g = g_ref[...]
    g = jax.nn.relu(jnp.dot(g, w1_ref[...], preferred_element_type=jnp.float32) + b1_ref[...])
    g = jax.nn.relu(jnp.dot(g, w2_ref[...], preferred_element_type=jnp.float32) + b2_ref[...])
    out_ref[...] = jnp.dot(g, w3_ref[...], preferred_element_type=jnp.float32) + b3_ref[...]


def kernel(x, edge_index, edge_attr, pos, batch, params):
    src = edge_index[0].astype(jnp.int32)
    dst = edge_index[1].astype(jnp.int32)
    perm = jnp.argsort(dst)
    dst_s = dst[perm]
    src_s = src[perm]
    ea_s = edge_attr[perm]
    ebounds = jnp.searchsorted(dst_s, jnp.arange(0, NPAD + NPW, NPW, dtype=jnp.int32)
                               ).astype(jnp.int32)
    ebounds = jnp.pad(ebounds, (0, 48 - NW - 1), constant_values=E)
    dst_s = jnp.pad(dst_s, (0, EPAD - E))
    src_s = jnp.pad(src_s, (0, EPAD - E))
    ea_s = jnp.pad(ea_s, ((0, EPAD - E), (0, 0)))
    dst_sf = dst_s.astype(jnp.float32)
    ebounds_f = ebounds.astype(jnp.float32)

    h = x @ params['node_emb_W'] + params['node_emb_b']
    freq = jnp.linspace(0.0, 10.0, NFREQ, dtype=jnp.float32)
    enc = []
    for i in range(3):
        ang = pos[:, i:i + 1] * freq[None, :]
        enc.append(jnp.sin(ang))
        enc.append(jnp.cos(ang))
    pe = jnp.concatenate(enc, axis=1) @ params['pos_W'] + params['pos_b']
    h = h + pe

    EW = params['edge_emb_W']
    ebias = params['edge_emb_b']
    scale = 1.0 / (float(C) ** 0.5)
    sel = jnp.kron(jnp.eye(H, dtype=jnp.float32),
                   jnp.full((C, 1), scale, jnp.float32))
    for l in range(L):
        p = params['layers'][l]
        F = EW @ p['We']
        g = ebias @ p['We'] + p['be']
        res = h
        w4 = jnp.concatenate([p['Wq'], p['Wk'], p['Wv'], p['Wskip']], axis=1)
        b4 = jnp.concatenate([p['bq'], p['bk'] + g, p['bv'] + g, p['bskip']])
        x4 = _qkvr(h, w4, b4.reshape(1, 4 * D))
        Q = x4[:, :D]
        K = x4[:, D:2 * D]
        V = x4[:, 2 * D:3 * D]
        r = x4[:, 3 * D:]
        qr, kr, vr = _gather(Q, K, V, dst_s, src_s)
        logits, vf = _edge_tc(qr, kr, vr, ea_s, F, sel)
        msg = _soft(logits.reshape(-1), vf.reshape(-1), dst_sf, ebounds_f)
        msg = msg.reshape(NPAD, D)[:N]
        wb = p['Wbeta'][:, 0]
        wbm = wb[:D] + wb[2 * D:]
        wbr = wb[D:2 * D] - wb[2 * D:]
        beta = jax.nn.sigmoid((msg @ wbm + r @ wbr + p['bbeta'])[:, None])
        out = beta * r + (1.0 - beta) * msg
        mu = out.mean(-1, keepdims=True)
        var = out.var(-1, keepdims=True)
        out = (out - mu) / jnp.sqrt(var + 1e-5) * p['ln_g'] + p['ln_b']
        out = jax.nn.relu(out)
        if l > 0:
            out = out + res
        h = out

    add = jax.ops.segment_sum(h, batch, num_segments=G)
    cnt = jax.ops.segment_sum(jnp.ones((N, 1), jnp.float32), batch, num_segments=G)
    mean = add / jnp.maximum(cnt, 1.0)
    mx = jax.ops.segment_max(h, batch, num_segments=G)
    mx = jnp.where(jnp.isfinite(mx), mx, 0.0)
    gfeat = jnp.concatenate([mean, mx, add], axis=1)
    out = pl.pallas_call(
        _readout_body,
        out_shape=jax.ShapeDtypeStruct((G, OUT), jnp.float32),
    )(gfeat, params['r_W1'], params['r_b1'], params['r_W2'], params['r_b2'],
      params['r_W3'], params['r_b3'])
    return out


# speedup vs baseline: 1.9295x; 1.0088x over previous
import functools

import jax
import jax.numpy as jnp
from jax import lax
from jax.experimental import pallas as pl
from jax.experimental.pallas import tpu as pltpu
from jax.experimental.pallas import tpu_sc as plsc

N = 10000
E = 160000
D = 256
H = 8
C = D // H
L = 6
ED = 4
OUT = 128
G = 64
NFREQ = D // 6

NW = 32
GCH = 64
EPAD = 163840
EPW = EPAD // NW
GITER = EPW // (2 * GCH)

NPW = 313
NPAD = NW * NPW
CE = 512
CE3 = 64
ACCW = NPW * D
MSW = 2512

_mesh = plsc.VectorSubcoreMesh(core_axis_name="c", subcore_axis_name="s")


def _gather_body(q_hbm, kv_hbm, di_hbm, si_hbm, qo, kvo,
                 idxd, idxs, qba, kvba, qbb, kvbb,
                 sqa, ska, sqb, skb,
                 wqa, wka, wqb, wkb):
    wid = lax.axis_index("c") * 16 + lax.axis_index("s")
    wbase = wid * EPW
    pltpu.sync_copy(di_hbm.at[pl.ds(wbase, EPW)], idxd)
    pltpu.sync_copy(si_hbm.at[pl.ds(wbase, EPW)], idxs)

    def it(i, carry):
        oa = 2 * i * GCH
        ob = oa + GCH
        ga = [
            pltpu.async_copy(q_hbm.at[idxd.at[pl.ds(oa, GCH)]], qba, sqa),
            pltpu.async_copy(kv_hbm.at[idxs.at[pl.ds(oa, GCH)]], kvba, ska),
        ]
        gb = [
            pltpu.async_copy(q_hbm.at[idxd.at[pl.ds(ob, GCH)]], qbb, sqb),
            pltpu.async_copy(kv_hbm.at[idxs.at[pl.ds(ob, GCH)]], kvbb, skb),
        ]
        for g in ga:
            g.wait()
        wa = [
            pltpu.async_copy(qba, qo.at[pl.ds(wbase + oa, GCH)], wqa),
            pltpu.async_copy(kvba, kvo.at[pl.ds(wbase + oa, GCH)], wka),
        ]
        for g in gb:
            g.wait()
        wb = [
            pltpu.async_copy(qbb, qo.at[pl.ds(wbase + ob, GCH)], wqb),
            pltpu.async_copy(kvbb, kvo.at[pl.ds(wbase + ob, GCH)], wkb),
        ]
        for w in wa + wb:
            w.wait()
        return carry

    lax.fori_loop(0, GITER, it, 0)


_gather = pl.kernel(
    _gather_body,
    out_type=[jax.ShapeDtypeStruct((EPAD, D), jnp.float32),
              jax.ShapeDtypeStruct((EPAD, 2 * D), jnp.float32)],
    mesh=_mesh,
    scratch_types=[
        pltpu.VMEM((EPW,), jnp.int32),
        pltpu.VMEM((EPW,), jnp.int32),
        pltpu.VMEM((GCH, D), jnp.float32),
        pltpu.VMEM((GCH, 2 * D), jnp.float32),
        pltpu.VMEM((GCH, D), jnp.float32),
        pltpu.VMEM((GCH, 2 * D), jnp.float32),
    ] + [pltpu.SemaphoreType.DMA] * 8,
)


def _extract48(ref, pos):
    total = jnp.float32(0)
    for j in range(3):
        v = ref[pl.ds(j * 16, 16)]
        lane = lax.broadcasted_iota(jnp.int32, (16,), 0) + (j * 16)
        total = total + jnp.sum(jnp.where(lane == pos, v, 0.0))
    return total.astype(jnp.int32)


def _soft_body(lg_hbm, vf_hbm, di_hbm, eb_hbm, msg_hbm,
               acc, msum, ssum, lbuf, dbuf, ebv,
               vbufa, lbufa, dbufa, abufa, vbufb, lbufb, dbufb, abufb,
               sva3, sla3, sda3, svb3, slb3, sdb3):
    wid = lax.axis_index("c") * 16 + lax.axis_index("s")
    n0 = wid * NPW
    iota = lax.broadcasted_iota(jnp.int32, (16,), 0)
    half = lax.shift_right_logical(iota, 3)
    lane7 = jnp.bitwise_and(iota, 7)
    n0f = jnp.float32(n0)
    pltpu.sync_copy(eb_hbm, ebv)
    e0 = _extract48(ebv, wid)
    e1 = _extract48(ebv, wid + 1)
    estart = (e0 // 8) * 8
    nch = (e1 - estart + CE - 1) // CE
    nch3 = (e1 - estart + CE3 - 1) // CE3

    zz = jnp.zeros((16,), jnp.float32)

    def _zacc(j, c):
        acc[pl.ds(j * 16, 16)] = zz
        return c
    lax.fori_loop(0, ACCW // 16, _zacc, 0)

    def _zs(j, c):
        ssum[pl.ds(j * 16, 16)] = zz
        return c
    lax.fori_loop(0, MSW // 16, _zs, 0)

    def _edge_vec(base, r):
        lg = lbuf[pl.ds(r * 16, 16)]
        eg = base + 2 * r + half
        mk = (eg >= e0) & (eg < e1)
        dv = plsc.load_gather(dbuf, [2 * r + half], mask=mk)
        dl = jnp.clip(dv - n0f, 0.0, float(NPW - 1))
        fidx = (dl * 8.0).astype(jnp.int32) + lane7
        return lg, fidx, mk

    def p1_chunk(c, carry):
        base = estart + c * CE
        pltpu.sync_copy(lg_hbm.at[pl.ds(base * 8, CE * 8)], lbuf)
        pltpu.sync_copy(di_hbm.at[pl.ds(base, CE)], dbuf)

        def it(r, cc):
            lg, fidx, mk = _edge_vec(base, r)
            plsc.store_scatter(msum, [fidx], lg, mask=mk)
            return cc
        lax.fori_loop(0, CE // 2, it, 0)
        return carry
    lax.fori_loop(0, nch, p1_chunk, 0)

    def p2_chunk(c, carry):
        base = estart + c * CE
        pltpu.sync_copy(lg_hbm.at[pl.ds(base * 8, CE * 8)], lbuf)
        pltpu.sync_copy(di_hbm.at[pl.ds(base, CE)], dbuf)

        def it(r, cc):
            lg, fidx, mk = _edge_vec(base, r)
            m = plsc.load_gather(msum, [fidx], mask=mk)
            ex = jnp.exp(lg - m)
            plsc.addupdate_scatter(ssum, [fidx], ex, mask=mk & (iota < 8))
            plsc.addupdate_scatter(ssum, [fidx], ex, mask=mk & (iota >= 8))
            return cc
        lax.fori_loop(0, CE // 2, it, 0)
        return carry
    lax.fori_loop(0, nch, p2_chunk, 0)

    def _proc3(base, vbuf3, lbuf3, dbuf3, abuf3):
        def ita(r, cc):
            lg = lbuf3[pl.ds(r * 16, 16)]
            eg = base + 2 * r + half
            mk = (eg >= e0) & (eg < e1)
            dv = plsc.load_gather(dbuf3, [2 * r + half], mask=mk)
            dl = jnp.clip(dv - n0f, 0.0, float(NPW - 1))
            fidx = (dl * 8.0).astype(jnp.int32) + lane7
            m = plsc.load_gather(msum, [fidx], mask=mk)
            s = plsc.load_gather(ssum, [fidx], mask=mk)
            abuf3[pl.ds(r * 16, 16)] = jnp.exp(lg - m) / (s + 1e-16)
            return cc
        lax.fori_loop(0, CE3 * 8 // 16, ita, 0)

        def ite(e, cc):
            eg = jnp.full((16,), base + e, jnp.int32)
            mk = (eg >= e0) & (eg < e1)
            dv = plsc.load_gather(dbuf3, [jnp.full((16,), e, jnp.int32)], mask=mk)
            bl = (jnp.clip(dv - n0f, 0.0, float(NPW - 1)) * float(D)).astype(jnp.int32) + iota
            for hh in range(H):
                av = plsc.load_gather(abuf3, [jnp.full((16,), e * 8 + hh, jnp.int32)], mask=mk)
                for f in (2 * hh, 2 * hh + 1):
                    vv = vbuf3[pl.ds(e * D + f * 16, 16)]
                    plsc.addupdate_scatter(acc, [bl + f * 16], vv * av, mask=mk)
            return cc
        lax.fori_loop(0, CE3, ite, 0)

    npair = (e1 - estart + 2 * CE3 - 1) // (2 * CE3)

    def p3_pair(j, carry):
        ba = estart + j * 2 * CE3
        bb = ba + CE3
        da = [
            pltpu.async_copy(vf_hbm.at[pl.ds(ba * D, CE3 * D)], vbufa, sva3),
            pltpu.async_copy(lg_hbm.at[pl.ds(ba * 8, CE3 * 8)], lbufa, sla3),
            pltpu.async_copy(di_hbm.at[pl.ds(ba, CE3)], dbufa, sda3),
        ]
        db = [
            pltpu.async_copy(vf_hbm.at[pl.ds(bb * D, CE3 * D)], vbufb, svb3),
            pltpu.async_copy(lg_hbm.at[pl.ds(bb * 8, CE3 * 8)], lbufb, slb3),
            pltpu.async_copy(di_hbm.at[pl.ds(bb, CE3)], dbufb, sdb3),
        ]
        for d in da:
            d.wait()
        _proc3(ba, vbufa, lbufa, dbufa, abufa)
        for d in db:
            d.wait()
        _proc3(bb, vbufb, lbufb, dbufb, abufb)
        return carry
    lax.fori_loop(0, npair, p3_pair, 0)

    pltpu.sync_copy(acc, msg_hbm.at[pl.ds(n0 * D, NPW * D)])


_soft = pl.kernel(
    _soft_body,
    out_type=jax.ShapeDtypeStruct((NPAD * D,), jnp.float32),
    mesh=_mesh,
    compiler_params=pltpu.CompilerParams(needs_layout_passes=False),
    scratch_types=[
        pltpu.VMEM((ACCW,), jnp.float32),
        pltpu.VMEM((MSW,), jnp.float32),
        pltpu.VMEM((MSW,), jnp.float32),
        pltpu.VMEM((CE * 8,), jnp.float32),
        pltpu.VMEM((CE,), jnp.float32),
        pltpu.VMEM((48,), jnp.float32),
        pltpu.VMEM((CE3 * D,), jnp.float32),
        pltpu.VMEM((CE3 * 8,), jnp.float32),
        pltpu.VMEM((CE3,), jnp.float32),
        pltpu.VMEM((CE3 * 8,), jnp.float32),
        pltpu.VMEM((CE3 * D,), jnp.float32),
        pltpu.VMEM((CE3 * 8,), jnp.float32),
        pltpu.VMEM((CE3,), jnp.float32),
        pltpu.VMEM((CE3 * 8,), jnp.float32),
    ] + [pltpu.SemaphoreType.DMA] * 6,
)


BE = 2048
BN = 400


def _edge_body(qr_ref, kv_ref, ea_ref, f_ref, sel_ref, lg_ref, vf_ref):
    u = jnp.dot(ea_ref[...], f_ref[...], preferred_element_type=jnp.float32)
    kf = kv_ref[:, :D] + u
    lg_ref[...] = jnp.dot(qr_ref[...] * kf, sel_ref[...],
                          preferred_element_type=jnp.float32)
    vf_ref[...] = kv_ref[:, D:] + u


_edge_tc = pl.pallas_call(
    _edge_body,
    grid=(EPAD // BE,),
    in_specs=[
        pl.BlockSpec((BE, D), lambda i: (i, 0)),
        pl.BlockSpec((BE, 2 * D), lambda i: (i, 0)),
        pl.BlockSpec((BE, ED), lambda i: (i, 0)),
        pl.BlockSpec((ED, D), lambda i: (0, 0)),
        pl.BlockSpec((D, H), lambda i: (0, 0)),
    ],
    out_specs=[
        pl.BlockSpec((BE, H), lambda i: (i, 0)),
        pl.BlockSpec((BE, D), lambda i: (i, 0)),
    ],
    out_shape=[
        jax.ShapeDtypeStruct((EPAD, H), jnp.float32),
        jax.ShapeDtypeStruct((EPAD, D), jnp.float32),
    ],
)


def _qkvr_body(h_ref, w_ref, b_ref, o_ref):
    o_ref[...] = jnp.dot(h_ref[...], w_ref[...],
                         preferred_element_type=jnp.float32) + b_ref[...]


_qkvr = pl.pallas_call(
    _qkvr_body,
    grid=(N // BN,),
    in_specs=[
        pl.BlockSpec((BN, D), lambda i: (i, 0)),
        pl.BlockSpec((D, 4 * D), lambda i: (0, 0)),
        pl.BlockSpec((1, 4 * D), lambda i: (0, 0)),
    ],
    out_specs=pl.BlockSpec((BN, 4 * D), lambda i: (i, 0)),
    out_shape=jax.ShapeDtypeStruct((N, 4 * D), jnp.float32),
)


def _readout_body(g_ref, w1_ref, b1_ref, w2_ref, b2_ref, w3_ref, b3_ref, out_ref):
    g = g_ref[...]
    g = jax.nn.relu(jnp.dot(g, w1_ref[...], preferred_element_type=jnp.float32) + b1_ref[...])
    g = jax.nn.relu(jnp.dot(g, w2_ref[...], preferred_element_type=jnp.float32) + b2_ref[...])
    out_ref[...] = jnp.dot(g, w3_ref[...], preferred_element_type=jnp.float32) + b3_ref[...]


def kernel(x, edge_index, edge_attr, pos, batch, params):
    src = edge_index[0].astype(jnp.int32)
    dst = edge_index[1].astype(jnp.int32)
    perm = jnp.argsort(dst)
    dst_s = dst[perm]
    src_s = src[perm]
    ea_s = edge_attr[perm]
    ebounds = jnp.searchsorted(dst_s, jnp.arange(0, NPAD + NPW, NPW, dtype=jnp.int32)
                               ).astype(jnp.int32)
    ebounds = jnp.pad(ebounds, (0, 48 - NW - 1), constant_values=E)
    dst_s = jnp.pad(dst_s, (0, EPAD - E))
    src_s = jnp.pad(src_s, (0, EPAD - E))
    ea_s = jnp.pad(ea_s, ((0, EPAD - E), (0, 0)))
    dst_sf = dst_s.astype(jnp.float32)
    ebounds_f = ebounds.astype(jnp.float32)

    h = x @ params['node_emb_W'] + params['node_emb_b']
    freq = jnp.linspace(0.0, 10.0, NFREQ, dtype=jnp.float32)
    enc = []
    for i in range(3):
        ang = pos[:, i:i + 1] * freq[None, :]
        enc.append(jnp.sin(ang))
        enc.append(jnp.cos(ang))
    pe = jnp.concatenate(enc, axis=1) @ params['pos_W'] + params['pos_b']
    h = h + pe

    EW = params['edge_emb_W']
    ebias = params['edge_emb_b']
    scale = 1.0 / (float(C) ** 0.5)
    sel = jnp.kron(jnp.eye(H, dtype=jnp.float32),
                   jnp.full((C, 1), scale, jnp.float32))
    for l in range(L):
        p = params['layers'][l]
        F = EW @ p['We']
        g = ebias @ p['We'] + p['be']
        res = h
        w4 = jnp.concatenate([p['Wq'], p['Wk'], p['Wv'], p['Wskip']], axis=1)
        b4 = jnp.concatenate([p['bq'], p['bk'] + g, p['bv'] + g, p['bskip']])
        x4 = _qkvr(h, w4, b4.reshape(1, 4 * D))
        Q = x4[:, :D]
        KV = x4[:, D:3 * D]
        r = x4[:, 3 * D:]
        qr, kvr = _gather(Q, KV, dst_s, src_s)
        logits, vf = _edge_tc(qr, kvr, ea_s, F, sel)
        msg = _soft(logits.reshape(-1), vf.reshape(-1), dst_sf, ebounds_f)
        msg = msg.reshape(NPAD, D)[:N]
        wb = p['Wbeta'][:, 0]
        wbm = wb[:D] + wb[2 * D:]
        wbr = wb[D:2 * D] - wb[2 * D:]
        beta = jax.nn.sigmoid((msg @ wbm + r @ wbr + p['bbeta'])[:, None])
        out = beta * r + (1.0 - beta) * msg
        mu = out.mean(-1, keepdims=True)
        var = out.var(-1, keepdims=True)
        out = (out - mu) / jnp.sqrt(var + 1e-5) * p['ln_g'] + p['ln_b']
        out = jax.nn.relu(out)
        if l > 0:
            out = out + res
        h = out

    add = jax.ops.segment_sum(h, batch, num_segments=G)
    cnt = jax.ops.segment_sum(jnp.ones((N, 1), jnp.float32), batch, num_segments=G)
    mean = add / jnp.maximum(cnt, 1.0)
    mx = jax.ops.segment_max(h, batch, num_segments=G)
    mx = jnp.where(jnp.isfinite(mx), mx, 0.0)
    gfeat = jnp.concatenate([mean, mx, add], axis=1)
    out = pl.pallas_call(
        _readout_body,
        out_shape=jax.ShapeDtypeStruct((G, OUT), jnp.float32),
    )(gfeat, params['r_W1'], params['r_b1'], params['r_W2'], params['r_b2'],
      params['r_W3'], params['r_b3'])
    return out


# speedup vs baseline: 1.9546x; 1.0130x over previous
import functools

import jax
import jax.numpy as jnp
from jax import lax
from jax.experimental import pallas as pl
from jax.experimental.pallas import tpu as pltpu
from jax.experimental.pallas import tpu_sc as plsc

N = 10000
E = 160000
D = 256
H = 8
C = D // H
L = 6
ED = 4
OUT = 128
G = 64
NFREQ = D // 6

NW = 32
GCH = 64
EPAD = 163840
EPW = EPAD // NW
GITER = EPW // (2 * GCH)

NPW = 313
NPAD = NW * NPW
CE = 512
CE3 = 64
ACCW = NPW * D
MSW = 2512

_mesh = plsc.VectorSubcoreMesh(core_axis_name="c", subcore_axis_name="s")


def _gather_body(q_hbm, kv_hbm, di_hbm, si_hbm, qo, kvo,
                 idxd, idxs, qba, kvba, qbb, kvbb,
                 sqa, ska, sqb, skb,
                 wqa, wka, wqb, wkb):
    wid = lax.axis_index("c") * 16 + lax.axis_index("s")
    wbase = wid * EPW
    pltpu.sync_copy(di_hbm.at[pl.ds(wbase, EPW)], idxd)
    pltpu.sync_copy(si_hbm.at[pl.ds(wbase, EPW)], idxs)

    def it(i, carry):
        oa = 2 * i * GCH
        ob = oa + GCH
        ga = [
            pltpu.async_copy(q_hbm.at[idxd.at[pl.ds(oa, GCH)]], qba, sqa),
            pltpu.async_copy(kv_hbm.at[idxs.at[pl.ds(oa, GCH)]], kvba, ska),
        ]
        gb = [
            pltpu.async_copy(q_hbm.at[idxd.at[pl.ds(ob, GCH)]], qbb, sqb),
            pltpu.async_copy(kv_hbm.at[idxs.at[pl.ds(ob, GCH)]], kvbb, skb),
        ]
        for g in ga:
            g.wait()
        wa = [
            pltpu.async_copy(qba, qo.at[pl.ds(wbase + oa, GCH)], wqa),
            pltpu.async_copy(kvba, kvo.at[pl.ds(wbase + oa, GCH)], wka),
        ]
        for g in gb:
            g.wait()
        wb = [
            pltpu.async_copy(qbb, qo.at[pl.ds(wbase + ob, GCH)], wqb),
            pltpu.async_copy(kvbb, kvo.at[pl.ds(wbase + ob, GCH)], wkb),
        ]
        for w in wa + wb:
            w.wait()
        return carry

    lax.fori_loop(0, GITER, it, 0)


_gather = pl.kernel(
    _gather_body,
    out_type=[jax.ShapeDtypeStruct((EPAD, D), jnp.float32),
              jax.ShapeDtypeStruct((EPAD, 2 * D), jnp.float32)],
    mesh=_mesh,
    scratch_types=[
        pltpu.VMEM((EPW,), jnp.int32),
        pltpu.VMEM((EPW,), jnp.int32),
        pltpu.VMEM((GCH, D), jnp.float32),
        pltpu.VMEM((GCH, 2 * D), jnp.float32),
        pltpu.VMEM((GCH, D), jnp.float32),
        pltpu.VMEM((GCH, 2 * D), jnp.float32),
    ] + [pltpu.SemaphoreType.DMA] * 8,
)


def _extract48(ref, pos):
    total = jnp.float32(0)
    for j in range(3):
        v = ref[pl.ds(j * 16, 16)]
        lane = lax.broadcasted_iota(jnp.int32, (16,), 0) + (j * 16)
        total = total + jnp.sum(jnp.where(lane == pos, v, 0.0))
    return total.astype(jnp.int32)


def _soft_body(lg_hbm, vf_hbm, di_hbm, eb_hbm, msg_hbm,
               acc, msum, ssum, lbuf, dbuf, ebv,
               vbufa, lbufa, dbufa, abufa, vbufb, lbufb, dbufb, abufb,
               sva3, sla3, sda3, svb3, slb3, sdb3):
    wid = lax.axis_index("c") * 16 + lax.axis_index("s")
    n0 = wid * NPW
    iota = lax.broadcasted_iota(jnp.int32, (16,), 0)
    half = lax.shift_right_logical(iota, 3)
    lane7 = jnp.bitwise_and(iota, 7)
    n0f = jnp.float32(n0)
    pltpu.sync_copy(eb_hbm, ebv)
    e0 = _extract48(ebv, wid)
    e1 = _extract48(ebv, wid + 1)
    estart = (e0 // 8) * 8
    nch = (e1 - estart + CE - 1) // CE
    nch3 = (e1 - estart + CE3 - 1) // CE3

    zz = jnp.zeros((16,), jnp.float32)

    def _zacc(j, c):
        acc[pl.ds(j * 16, 16)] = zz
        return c
    lax.fori_loop(0, ACCW // 16, _zacc, 0)

    def _zs(j, c):
        ssum[pl.ds(j * 16, 16)] = zz
        return c
    lax.fori_loop(0, MSW // 16, _zs, 0)

    def _edge_vec(base, r):
        lg = lbuf[pl.ds(r * 16, 16)]
        eg = base + 2 * r + half
        mk = (eg >= e0) & (eg < e1)
        dv = plsc.load_gather(dbuf, [2 * r + half], mask=mk)
        dl = jnp.clip(dv - n0f, 0.0, float(NPW - 1))
        fidx = (dl * 8.0).astype(jnp.int32) + lane7
        return lg, fidx, mk

    def p1_chunk(c, carry):
        base = estart + c * CE
        pltpu.sync_copy(lg_hbm.at[pl.ds(base * 8, CE * 8)], lbuf)
        pltpu.sync_copy(di_hbm.at[pl.ds(base, CE)], dbuf)

        def it(r, cc):
            lg, fidx, mk = _edge_vec(base, r)
            plsc.store_scatter(msum, [fidx], lg, mask=mk)
            return cc
        lax.fori_loop(0, CE // 2, it, 0)
        return carry
    lax.fori_loop(0, nch, p1_chunk, 0)

    def p2_chunk(c, carry):
        base = estart + c * CE
        pltpu.sync_copy(lg_hbm.at[pl.ds(base * 8, CE * 8)], lbuf)
        pltpu.sync_copy(di_hbm.at[pl.ds(base, CE)], dbuf)

        def it(r, cc):
            lg, fidx, mk = _edge_vec(base, r)
            m = plsc.load_gather(msum, [fidx], mask=mk)
            ex = jnp.exp(lg - m)
            plsc.addupdate_scatter(ssum, [fidx], ex, mask=mk & (iota < 8))
            plsc.addupdate_scatter(ssum, [fidx], ex, mask=mk & (iota >= 8))
            return cc
        lax.fori_loop(0, CE // 2, it, 0)
        return carry
    lax.fori_loop(0, nch, p2_chunk, 0)

    def _proc3(base, vbuf3, lbuf3, dbuf3, abuf3):
        def ita(r, cc):
            lg = lbuf3[pl.ds(r * 16, 16)]
            eg = base + 2 * r + half
            mk = (eg >= e0) & (eg < e1)
            dv = plsc.load_gather(dbuf3, [2 * r + half], mask=mk)
            dl = jnp.clip(dv - n0f, 0.0, float(NPW - 1))
            fidx = (dl * 8.0).astype(jnp.int32) + lane7
            m = plsc.load_gather(msum, [fidx], mask=mk)
            s = plsc.load_gather(ssum, [fidx], mask=mk)
            abuf3[pl.ds(r * 16, 16)] = jnp.exp(lg - m) / (s + 1e-16)
            return cc
        lax.fori_loop(0, CE3 * 8 // 16, ita, 0)

        def ite(e, cc):
            eg = jnp.full((16,), base + e, jnp.int32)
            mk = (eg >= e0) & (eg < e1)
            dv = plsc.load_gather(dbuf3, [jnp.full((16,), e, jnp.int32)], mask=mk)
            bl = (jnp.clip(dv - n0f, 0.0, float(NPW - 1)) * float(D)).astype(jnp.int32) + iota
            for hh in range(H):
                av = plsc.load_gather(abuf3, [jnp.full((16,), e * 8 + hh, jnp.int32)], mask=mk)
                for f in (2 * hh, 2 * hh + 1):
                    vv = vbuf3[pl.ds(e * D + f * 16, 16)]
                    plsc.addupdate_scatter(acc, [bl + f * 16], vv * av, mask=mk)
            return cc
        lax.fori_loop(0, CE3, ite, 0)

    npair = (e1 - estart + 2 * CE3 - 1) // (2 * CE3)

    def p3_pair(j, carry):
        ba = estart + j * 2 * CE3
        bb = ba + CE3
        da = [
            pltpu.async_copy(vf_hbm.at[pl.ds(ba * D, CE3 * D)], vbufa, sva3),
            pltpu.async_copy(lg_hbm.at[pl.ds(ba * 8, CE3 * 8)], lbufa, sla3),
            pltpu.async_copy(di_hbm.at[pl.ds(ba, CE3)], dbufa, sda3),
        ]
        db = [
            pltpu.async_copy(vf_hbm.at[pl.ds(bb * D, CE3 * D)], vbufb, svb3),
            pltpu.async_copy(lg_hbm.at[pl.ds(bb * 8, CE3 * 8)], lbufb, slb3),
            pltpu.async_copy(di_hbm.at[pl.ds(bb, CE3)], dbufb, sdb3),
        ]
        for d in da:
            d.wait()
        _proc3(ba, vbufa, lbufa, dbufa, abufa)
        for d in db:
            d.wait()
        _proc3(bb, vbufb, lbufb, dbufb, abufb)
        return carry
    lax.fori_loop(0, npair, p3_pair, 0)

    pltpu.sync_copy(acc, msg_hbm.at[pl.ds(n0 * D, NPW * D)])


_soft = pl.kernel(
    _soft_body,
    out_type=jax.ShapeDtypeStruct((NPAD * D,), jnp.float32),
    mesh=_mesh,
    compiler_params=pltpu.CompilerParams(needs_layout_passes=False),
    scratch_types=[
        pltpu.VMEM((ACCW,), jnp.float32),
        pltpu.VMEM((MSW,), jnp.float32),
        pltpu.VMEM((MSW,), jnp.float32),
        pltpu.VMEM((CE * 8,), jnp.float32),
        pltpu.VMEM((CE,), jnp.float32),
        pltpu.VMEM((48,), jnp.float32),
        pltpu.VMEM((CE3 * D,), jnp.float32),
        pltpu.VMEM((CE3 * 8,), jnp.float32),
        pltpu.VMEM((CE3,), jnp.float32),
        pltpu.VMEM((CE3 * 8,), jnp.float32),
        pltpu.VMEM((CE3 * D,), jnp.float32),
        pltpu.VMEM((CE3 * 8,), jnp.float32),
        pltpu.VMEM((CE3,), jnp.float32),
        pltpu.VMEM((CE3 * 8,), jnp.float32),
    ] + [pltpu.SemaphoreType.DMA] * 6,
)


BE = 2048
BN = 400


def _edge_body(qr_ref, kv_ref, ea_ref, f_ref, sel_ref, lg_ref, vf_ref):
    u = jnp.dot(ea_ref[...], f_ref[...], preferred_element_type=jnp.float32)
    kf = kv_ref[:, :D] + u
    lg_ref[...] = jnp.dot(qr_ref[...] * kf, sel_ref[...],
                          preferred_element_type=jnp.float32)
    vf_ref[...] = kv_ref[:, D:] + u


_edge_tc = pl.pallas_call(
    _edge_body,
    grid=(EPAD // BE,),
    in_specs=[
        pl.BlockSpec((BE, D), lambda i: (i, 0)),
        pl.BlockSpec((BE, 2 * D), lambda i: (i, 0)),
        pl.BlockSpec((BE, ED), lambda i: (i, 0)),
        pl.BlockSpec((ED, D), lambda i: (0, 0)),
        pl.BlockSpec((D, H), lambda i: (0, 0)),
    ],
    out_specs=[
        pl.BlockSpec((BE, H), lambda i: (i, 0)),
        pl.BlockSpec((BE, D), lambda i: (i, 0)),
    ],
    out_shape=[
        jax.ShapeDtypeStruct((EPAD, H), jnp.float32),
        jax.ShapeDtypeStruct((EPAD, D), jnp.float32),
    ],
)


def _qkvr_body(h_ref, w_ref, b_ref, o_ref):
    o_ref[...] = jnp.dot(h_ref[...], w_ref[...],
                         preferred_element_type=jnp.float32) + b_ref[...]


_qkvr = pl.pallas_call(
    _qkvr_body,
    grid=(N // BN,),
    in_specs=[
        pl.BlockSpec((BN, D), lambda i: (i, 0)),
        pl.BlockSpec((D, 4 * D), lambda i: (0, 0)),
        pl.BlockSpec((1, 4 * D), lambda i: (0, 0)),
    ],
    out_specs=pl.BlockSpec((BN, 4 * D), lambda i: (i, 0)),
    out_shape=jax.ShapeDtypeStruct((N, 4 * D), jnp.float32),
)


CHP = 64
GPW = G // NW


def _extract80(ref, pos):
    total = jnp.float32(0)
    for j in range(5):
        v = ref[pl.ds(j * 16, 16)]
        lane = lax.broadcasted_iota(jnp.int32, (16,), 0) + (j * 16)
        total = total + jnp.sum(jnp.where(lane == pos, v, 0.0))
    return total.astype(jnp.int32)


def _pool_body(h_hbm, gb_hbm, add_hbm, mx_hbm, adda, mxa, hbuf, gbv, sem):
    wid = lax.axis_index("c") * 16 + lax.axis_index("s")
    pltpu.sync_copy(gb_hbm, gbv)
    zz = jnp.zeros((16,), jnp.float32)
    ninf = jnp.full((16,), -jnp.inf, jnp.float32)

    def _zi(j, c):
        adda[pl.ds(j * 16, 16)] = zz
        mxa[pl.ds(j * 16, 16)] = ninf
        return c
    lax.fori_loop(0, 2 * D // 16, _zi, 0)

    for g in range(GPW):
        gid = wid * GPW + g
        n0g = _extract80(gbv, gid)
        n1g = _extract80(gbv, gid + 1)
        nchp = (n1g - n0g + CHP - 1) // CHP

        def chunk(c, carry):
            base = n0g + c * CHP
            pltpu.sync_copy(h_hbm.at[pl.ds(base * D, CHP * D)], hbuf)
            rows = jnp.minimum(CHP, n1g - n0g - c * CHP)

            def row(i, cc):
                for f in range(D // 16):
                    v = hbuf[pl.ds(i * D + f * 16, 16)]
                    o = g * D + f * 16
                    adda[pl.ds(o, 16)] = adda[pl.ds(o, 16)] + v
                    mxa[pl.ds(o, 16)] = jnp.maximum(mxa[pl.ds(o, 16)], v)
                return cc
            lax.fori_loop(0, rows, row, 0)
            return carry
        lax.fori_loop(0, nchp, chunk, 0)

    pltpu.sync_copy(adda, add_hbm.at[pl.ds(wid * GPW * D, GPW * D)])
    pltpu.sync_copy(mxa, mx_hbm.at[pl.ds(wid * GPW * D, GPW * D)])


_pool = pl.kernel(
    _pool_body,
    out_type=[jax.ShapeDtypeStruct((G * D,), jnp.float32)] * 2,
    mesh=_mesh,
    compiler_params=pltpu.CompilerParams(needs_layout_passes=False),
    scratch_types=[
        pltpu.VMEM((GPW * D,), jnp.float32),
        pltpu.VMEM((GPW * D,), jnp.float32),
        pltpu.VMEM((CHP * D,), jnp.float32),
        pltpu.VMEM((80,), jnp.float32),
        pltpu.SemaphoreType.DMA,
    ],
)


def _readout_body(g_ref, w1_ref, b1_ref, w2_ref, b2_ref, w3_ref, b3_ref, out_ref):
    g = g_ref[...]
    g = jax.nn.relu(jnp.dot(g, w1_ref[...], preferred_element_type=jnp.float32) + b1_ref[...])
    g = jax.nn.relu(jnp.dot(g, w2_ref[...], preferred_element_type=jnp.float32) + b2_ref[...])
    out_ref[...] = jnp.dot(g, w3_ref[...], preferred_element_type=jnp.float32) + b3_ref[...]


def kernel(x, edge_index, edge_attr, pos, batch, params):
    src = edge_index[0].astype(jnp.int32)
    dst = edge_index[1].astype(jnp.int32)
    perm = jnp.argsort(dst)
    dst_s = dst[perm]
    src_s = src[perm]
    ea_s = edge_attr[perm]
    ebounds = jnp.searchsorted(dst_s, jnp.arange(0, NPAD + NPW, NPW, dtype=jnp.int32)
                               ).astype(jnp.int32)
    ebounds = jnp.pad(ebounds, (0, 48 - NW - 1), constant_values=E)
    dst_s = jnp.pad(dst_s, (0, EPAD - E))
    src_s = jnp.pad(src_s, (0, EPAD - E))
    ea_s = jnp.pad(ea_s, ((0, EPAD - E), (0, 0)))
    dst_sf = dst_s.astype(jnp.float32)
    ebounds_f = ebounds.astype(jnp.float32)

    h = x @ params['node_emb_W'] + params['node_emb_b']
    freq = jnp.linspace(0.0, 10.0, NFREQ, dtype=jnp.float32)
    enc = []
    for i in range(3):
        ang = pos[:, i:i + 1] * freq[None, :]
        enc.append(jnp.sin(ang))
        enc.append(jnp.cos(ang))
    pe = jnp.concatenate(enc, axis=1) @ params['pos_W'] + params['pos_b']
    h = h + pe

    EW = params['edge_emb_W']
    ebias = params['edge_emb_b']
    scale = 1.0 / (float(C) ** 0.5)
    sel = jnp.kron(jnp.eye(H, dtype=jnp.float32),
                   jnp.full((C, 1), scale, jnp.float32))
    for l in range(L):
        p = params['layers'][l]
        F = EW @ p['We']
        g = ebias @ p['We'] + p['be']
        res = h
        w4 = jnp.concatenate([p['Wq'], p['Wk'], p['Wv'], p['Wskip']], axis=1)
        b4 = jnp.concatenate([p['bq'], p['bk'] + g, p['bv'] + g, p['bskip']])
        x4 = _qkvr(h, w4, b4.reshape(1, 4 * D))
        Q = x4[:, :D]
        KV = x4[:, D:3 * D]
        r = x4[:, 3 * D:]
        qr, kvr = _gather(Q, KV, dst_s, src_s)
        logits, vf = _edge_tc(qr, kvr, ea_s, F, sel)
        msg = _soft(logits.reshape(-1), vf.reshape(-1), dst_sf, ebounds_f)
        msg = msg.reshape(NPAD, D)[:N]
        wb = p['Wbeta'][:, 0]
        wbm = wb[:D] + wb[2 * D:]
        wbr = wb[D:2 * D] - wb[2 * D:]
        beta = jax.nn.sigmoid((msg @ wbm + r @ wbr + p['bbeta'])[:, None])
        out = beta * r + (1.0 - beta) * msg
        mu = out.mean(-1, keepdims=True)
        var = out.var(-1, keepdims=True)
        out = (out - mu) / jnp.sqrt(var + 1e-5) * p['ln_g'] + p['ln_b']
        out = jax.nn.relu(out)
        if l > 0:
            out = out + res
        h = out

    batch32 = batch.astype(jnp.int32)
    gbounds = jnp.searchsorted(batch32, jnp.arange(G + 1, dtype=jnp.int32)).astype(jnp.int32)
    cnt = (gbounds[1:] - gbounds[:-1]).astype(jnp.float32)[:, None]
    gb_f = jnp.pad(gbounds, (0, 80 - G - 1), constant_values=N).astype(jnp.float32)
    hflat = jnp.concatenate([h.reshape(-1), jnp.zeros((CHP * D,), jnp.float32)])
    addf, mxf = _pool(hflat, gb_f)
    add = addf.reshape(G, D)
    mx = mxf.reshape(G, D)
    mean = add / jnp.maximum(cnt, 1.0)
    mx = jnp.where(jnp.isfinite(mx), mx, 0.0)
    gfeat = jnp.concatenate([mean, mx, add], axis=1)
    out = pl.pallas_call(
        _readout_body,
        out_shape=jax.ShapeDtypeStruct((G, OUT), jnp.float32),
    )(gfeat, params['r_W1'], params['r_b1'], params['r_W2'], params['r_b2'],
      params['r_W3'], params['r_b3'])
    return out
